# Initial kernel scaffold; baseline (speedup 1.0000x reference)
import functools

import jax
import jax.numpy as jnp
from jax import lax
from jax.experimental import pallas as pl
from jax.experimental.pallas import tpu as pltpu
from jax.experimental.pallas import tpu_sc as plsc

N_NODES = 10000
N_FEAT = 128
WIN = 128
NC, NS = 2, 16
NW = NC * NS
ZCHUNK = 640

_MESH = plsc.VectorSubcoreMesh(
    core_axis_name="c", subcore_axis_name="s", num_cores=NC, num_subcores=NS
)


def _num_windows(E):
    return E // WIN


def _deg_body(E, edges_hbm, out_hbm, idx_v, ones_v, zb_v, cnt_sh):
    cid = lax.axis_index("c")
    sid = lax.axis_index("s")
    wid = cid * NS + sid

    for i in range(8):
        ones_v[pl.ds(i * 16, 16)] = jnp.full((16,), 1.0, jnp.float32)
    for i in range(ZCHUNK // 16):
        zb_v[pl.ds(i * 16, 16)] = jnp.zeros((16,), jnp.float32)

    start = sid * ZCHUNK

    @pl.when(sid < (N_NODES // ZCHUNK))
    def _():
        pltpu.sync_copy(zb_v, cnt_sh.at[pl.ds(start, ZCHUNK)])

    @pl.when(sid == (N_NODES // ZCHUNK))
    def _():
        rem = N_NODES - (N_NODES // ZCHUNK) * ZCHUNK
        pltpu.sync_copy(zb_v.at[pl.ds(0, rem)], cnt_sh.at[pl.ds(start, rem)])

    plsc.subcore_barrier()

    nwin = _num_windows(E)

    def body(k, _):
        wi = wid + NW * k

        @pl.when(wi < nwin)
        def _():
            base = wi * WIN
            pltpu.sync_copy(edges_hbm.at[1, pl.ds(base, WIN)], idx_v)
            pltpu.sync_copy(ones_v, cnt_sh.at[idx_v], add=True)

        return 0

    lax.fori_loop(0, (nwin + NW - 1) // NW, body, 0)
    plsc.subcore_barrier()

    @pl.when(sid < (N_NODES // ZCHUNK))
    def _():
        pltpu.sync_copy(cnt_sh.at[pl.ds(start, ZCHUNK)],
                        out_hbm.at[cid, pl.ds(start, ZCHUNK)])

    @pl.when(sid == (N_NODES // ZCHUNK))
    def _():
        rem = N_NODES - (N_NODES // ZCHUNK) * ZCHUNK
        pltpu.sync_copy(cnt_sh.at[pl.ds(start, rem)],
                        out_hbm.at[cid, pl.ds(start, rem)])


def _deg_counts(edge_index):
    E = edge_index.shape[1]
    k = pl.kernel(
        functools.partial(_deg_body, E),
        out_type=jax.ShapeDtypeStruct((NC, N_NODES), jnp.float32),
        mesh=_MESH,
        scratch_types=[
            pltpu.VMEM((WIN,), jnp.int32),
            pltpu.VMEM((WIN,), jnp.float32),
            pltpu.VMEM((ZCHUNK,), jnp.float32),
            pltpu.VMEM_SHARED((N_NODES,), jnp.float32),
        ],
    )
    return k(edge_index)


def _edge_body(E, hs_hbm, edges_hbm, out_hbm, sidx_v, didx_v, rows_v, zb_v,
               acc_sh, sem):
    cid = lax.axis_index("c")
    sid = lax.axis_index("s")
    wid = cid * NS + sid

    for i in range(16):
        for j in range(8):
            zb_v[i, pl.ds(j * 16, 16)] = jnp.zeros((16,), jnp.float32)

    rstart = sid * ZCHUNK

    def zbody(j, _):
        r0 = rstart + 16 * j

        @pl.when(r0 < N_NODES)
        def _():
            pltpu.sync_copy(zb_v, acc_sh.at[pl.ds(r0, 16)])

        return 0

    lax.fori_loop(0, ZCHUNK // 16, zbody, 0)
    plsc.subcore_barrier()

    nwin = _num_windows(E)

    def body(k, _):
        wi = wid + NW * k

        @pl.when(wi < nwin)
        def _():
            base = wi * WIN
            pltpu.sync_copy(edges_hbm.at[0, pl.ds(base, WIN)], sidx_v)
            pltpu.sync_copy(edges_hbm.at[1, pl.ds(base, WIN)], didx_v)
            pltpu.async_copy(hs_hbm.at[sidx_v], rows_v, sem).wait()
            pltpu.sync_copy(rows_v, acc_sh.at[didx_v], add=True)

        return 0

    lax.fori_loop(0, (nwin + NW - 1) // NW, body, 0)
    plsc.subcore_barrier()

    @pl.when(sid < (N_NODES // ZCHUNK))
    def _():
        pltpu.sync_copy(acc_sh.at[pl.ds(rstart, ZCHUNK)],
                        out_hbm.at[cid, pl.ds(rstart, ZCHUNK)])

    @pl.when(sid == (N_NODES // ZCHUNK))
    def _():
        rem = N_NODES - (N_NODES // ZCHUNK) * ZCHUNK
        pltpu.sync_copy(acc_sh.at[pl.ds(rstart, rem)],
                        out_hbm.at[cid, pl.ds(rstart, rem)])


def _edge_pass(hs, edge_index):
    E = edge_index.shape[1]
    k = pl.kernel(
        functools.partial(_edge_body, E),
        out_type=jax.ShapeDtypeStruct((NC, N_NODES, N_FEAT), jnp.float32),
        mesh=_MESH,
        scratch_types=[
            pltpu.VMEM((WIN,), jnp.int32),
            pltpu.VMEM((WIN,), jnp.int32),
            pltpu.VMEM((WIN, N_FEAT), jnp.float32),
            pltpu.VMEM((16, N_FEAT), jnp.float32),
            pltpu.VMEM_SHARED((N_NODES, N_FEAT), jnp.float32),
            pltpu.SemaphoreType.DMA,
        ],
    )
    return k(hs, edge_index)


_RB = 1000


def _dinv_from_counts(c_ref):
    deg = c_ref[0] + c_ref[1] + 1.0
    return lax.rsqrt(deg)


def _b1_body(x_ref, w_ref, c_ref, o_ref):
    dinv = _dinv_from_counts(c_ref)
    h = jnp.dot(x_ref[...], w_ref[...], preferred_element_type=jnp.float32,
                precision=lax.Precision.HIGHEST)
    o_ref[...] = h * dinv


def _matmul_scale(x, W, cnt3):
    return pl.pallas_call(
        _b1_body,
        grid=(N_NODES // _RB,),
        in_specs=[
            pl.BlockSpec((_RB, N_FEAT), lambda i: (i, 0)),
            pl.BlockSpec((N_FEAT, N_FEAT), lambda i: (0, 0)),
            pl.BlockSpec((NC, _RB, 1), lambda i: (0, i, 0)),
        ],
        out_specs=pl.BlockSpec((_RB, N_FEAT), lambda i: (i, 0)),
        out_shape=jax.ShapeDtypeStruct((N_NODES, N_FEAT), jnp.float32),
    )(x, W, cnt3)


def _b2_body(a_ref, hs_ref, c_ref, b_ref, w_ref, o_ref):
    dinv = _dinv_from_counts(c_ref)
    s = a_ref[0] + a_ref[1] + hs_ref[...]
    o = jax.nn.relu(dinv * s + b_ref[...])
    h2 = jnp.dot(o, w_ref[...], preferred_element_type=jnp.float32,
                 precision=lax.Precision.HIGHEST)
    o_ref[...] = h2 * dinv


def _conv_finish_matmul(acc, hs, cnt3, b2d, W):
    return pl.pallas_call(
        _b2_body,
        grid=(N_NODES // _RB,),
        in_specs=[
            pl.BlockSpec((NC, _RB, N_FEAT), lambda i: (0, i, 0)),
            pl.BlockSpec((_RB, N_FEAT), lambda i: (i, 0)),
            pl.BlockSpec((NC, _RB, 1), lambda i: (0, i, 0)),
            pl.BlockSpec((1, N_FEAT), lambda i: (0, 0)),
            pl.BlockSpec((N_FEAT, N_FEAT), lambda i: (0, 0)),
        ],
        out_specs=pl.BlockSpec((_RB, N_FEAT), lambda i: (i, 0)),
        out_shape=jax.ShapeDtypeStruct((N_NODES, N_FEAT), jnp.float32),
    )(acc, hs, cnt3, b2d, W)


def _b3_body(a_ref, hs_ref, c_ref, b_ref, o_ref):
    dinv = _dinv_from_counts(c_ref)
    s = a_ref[0] + a_ref[1] + hs_ref[...]
    o = jax.nn.relu(dinv * s + b_ref[...])
    part = jnp.sum(o, axis=0, keepdims=True)

    @pl.when(pl.program_id(0) == 0)
    def _():
        o_ref[...] = part

    @pl.when(pl.program_id(0) != 0)
    def _():
        o_ref[...] += part


def _conv_finish_pool(acc, hs, cnt3, b2d):
    return pl.pallas_call(
        _b3_body,
        grid=(N_NODES // _RB,),
        in_specs=[
            pl.BlockSpec((NC, _RB, N_FEAT), lambda i: (0, i, 0)),
            pl.BlockSpec((_RB, N_FEAT), lambda i: (i, 0)),
            pl.BlockSpec((NC, _RB, 1), lambda i: (0, i, 0)),
            pl.BlockSpec((1, N_FEAT), lambda i: (0, 0)),
        ],
        out_specs=pl.BlockSpec((1, N_FEAT), lambda i: (0, 0)),
        out_shape=jax.ShapeDtypeStruct((1, N_FEAT), jnp.float32),
    )(acc, hs, cnt3, b2d)


def _head_body(s_ref, w1_ref, b1_ref, w2_ref, b2_ref, o_ref):
    pooled = s_ref[...] * (1.0 / N_NODES)
    y = jax.nn.relu(
        jnp.dot(pooled, w1_ref[...], preferred_element_type=jnp.float32,
                precision=lax.Precision.HIGHEST) + b1_ref[...])
    z = jax.nn.relu(
        jnp.dot(y, w2_ref[...], preferred_element_type=jnp.float32,
                precision=lax.Precision.HIGHEST) + b2_ref[...])
    o_ref[...] = z


def _head(sums, Wl1, bl1_2d, Wl2, bl2_2d):
    return pl.pallas_call(
        _head_body,
        out_shape=jax.ShapeDtypeStruct((1, 1), jnp.float32),
    )(sums, Wl1, bl1_2d, Wl2, bl2_2d)


def kernel(x, edge_index, W1, b1, W2, b2, Wl1, bl1, Wl2, bl2):
    counts = _deg_counts(edge_index)
    cnt3 = counts.reshape(NC, N_NODES, 1)
    hs1 = _matmul_scale(x, W1, cnt3)
    acc1 = _edge_pass(hs1, edge_index)
    hs2 = _conv_finish_matmul(acc1, hs1, cnt3, b1.reshape(1, -1), W2)
    acc2 = _edge_pass(hs2, edge_index)
    sums = _conv_finish_pool(acc2, hs2, cnt3, b2.reshape(1, -1))
    out = _head(sums, Wl1, bl1.reshape(1, -1), Wl2, bl2.reshape(1, 1))
    return out


# speedup vs baseline: 16.3430x; 16.3430x over previous
import functools

import jax
import jax.numpy as jnp
from jax import lax
from jax.experimental import pallas as pl
from jax.experimental.pallas import tpu as pltpu
from jax.experimental.pallas import tpu_sc as plsc

N_NODES = 10000
N_FEAT = 128
WIN = 128
NC, NS = 2, 16
NW = NC * NS
ZCHUNK = 640

def _mesh():
    return plsc.VectorSubcoreMesh(
        core_axis_name="c", subcore_axis_name="s", num_cores=NC, num_subcores=NS
    )


def _num_windows(E):
    return E // WIN


def _deg_body(E, edges_hbm, out0_hbm, out1_hbm, idx_v, ones_v, zb_v, cnt_sh):
    cid = lax.axis_index("c")
    sid = lax.axis_index("s")
    wid = cid * NS + sid

    for i in range(8):
        ones_v[pl.ds(i * 16, 16)] = jnp.full((16,), 1.0, jnp.float32)
    for i in range(ZCHUNK // 16):
        zb_v[pl.ds(i * 16, 16)] = jnp.zeros((16,), jnp.float32)

    start = sid * ZCHUNK

    @pl.when(sid < (N_NODES // ZCHUNK))
    def _():
        pltpu.sync_copy(zb_v, cnt_sh.at[pl.ds(start, ZCHUNK)])

    @pl.when(sid == (N_NODES // ZCHUNK))
    def _():
        rem = N_NODES - (N_NODES // ZCHUNK) * ZCHUNK
        pltpu.sync_copy(zb_v.at[pl.ds(0, rem)], cnt_sh.at[pl.ds(start, rem)])

    plsc.subcore_barrier()

    nwin = _num_windows(E)

    def body(k, _):
        wi = wid + NW * k

        @pl.when(wi < nwin)
        def _():
            base = wi * WIN
            pltpu.sync_copy(edges_hbm.at[1, pl.ds(base, WIN)], idx_v)
            pltpu.sync_copy(ones_v, cnt_sh.at[idx_v], add=True)

        return 0

    lax.fori_loop(0, (nwin + NW - 1) // NW, body, 0)
    plsc.subcore_barrier()

    rem = N_NODES - (N_NODES // ZCHUNK) * ZCHUNK
    for c, out_hbm in ((0, out0_hbm), (1, out1_hbm)):
        @pl.when(cid == c)
        def _(out_hbm=out_hbm):
            @pl.when(sid < (N_NODES // ZCHUNK))
            def _():
                pltpu.sync_copy(cnt_sh.at[pl.ds(start, ZCHUNK)], zb_v)
                pltpu.sync_copy(zb_v, out_hbm.at[pl.ds(start, ZCHUNK)])

            @pl.when(sid == (N_NODES // ZCHUNK))
            def _():
                pltpu.sync_copy(cnt_sh.at[pl.ds(start, rem)],
                                zb_v.at[pl.ds(0, rem)])
                pltpu.sync_copy(zb_v.at[pl.ds(0, rem)],
                                out_hbm.at[pl.ds(start, rem)])


def _deg_counts(edge_index):
    E = edge_index.shape[1]
    k = pl.kernel(
        functools.partial(_deg_body, E),
        out_type=[jax.ShapeDtypeStruct((N_NODES,), jnp.float32),
                  jax.ShapeDtypeStruct((N_NODES,), jnp.float32)],
        mesh=_mesh(),
        scratch_types=[
            pltpu.VMEM((WIN,), jnp.int32),
            pltpu.VMEM((WIN,), jnp.float32),
            pltpu.VMEM((ZCHUNK,), jnp.float32),
            pltpu.VMEM_SHARED((N_NODES,), jnp.float32),
        ],
    )
    return k(edge_index)


def _edge_body(E, hs_hbm, edges_hbm, out0_hbm, out1_hbm, sidx_v, didx_v,
               rows_v, zb_v, acc_sh, sem):
    cid = lax.axis_index("c")
    sid = lax.axis_index("s")
    wid = cid * NS + sid

    for i in range(16):
        for j in range(8):
            zb_v[i, pl.ds(j * 16, 16)] = jnp.zeros((16,), jnp.float32)

    rstart = sid * ZCHUNK

    def zbody(j, _):
        r0 = rstart + 16 * j

        @pl.when(r0 < N_NODES)
        def _():
            pltpu.sync_copy(zb_v, acc_sh.at[pl.ds(r0, 16)])

        return 0

    lax.fori_loop(0, ZCHUNK // 16, zbody, 0)
    plsc.subcore_barrier()

    nwin = _num_windows(E)

    def body(k, _):
        wi = wid + NW * k

        @pl.when(wi < nwin)
        def _():
            base = wi * WIN
            pltpu.sync_copy(edges_hbm.at[0, pl.ds(base, WIN)], sidx_v)
            pltpu.sync_copy(edges_hbm.at[1, pl.ds(base, WIN)], didx_v)
            pltpu.async_copy(hs_hbm.at[sidx_v], rows_v, sem).wait()
            pltpu.sync_copy(rows_v, acc_sh.at[didx_v], add=True)

        return 0

    lax.fori_loop(0, (nwin + NW - 1) // NW, body, 0)
    plsc.subcore_barrier()

    nchunk_full = N_NODES // WIN
    tail_rows = N_NODES - nchunk_full * WIN

    for c, out_hbm in ((0, out0_hbm), (1, out1_hbm)):
        @pl.when(cid == c)
        def _(out_hbm=out_hbm):
            def wbody(k, _):
                ci = sid + NS * k
                r0 = ci * WIN

                @pl.when(ci < nchunk_full)
                def _():
                    pltpu.sync_copy(acc_sh.at[pl.ds(r0, WIN)], rows_v)
                    pltpu.sync_copy(rows_v, out_hbm.at[pl.ds(r0, WIN)])

                @pl.when(ci == nchunk_full)
                def _():
                    pltpu.sync_copy(acc_sh.at[pl.ds(r0, tail_rows)],
                                    rows_v.at[pl.ds(0, tail_rows)])
                    pltpu.sync_copy(rows_v.at[pl.ds(0, tail_rows)],
                                    out_hbm.at[pl.ds(r0, tail_rows)])

                return 0

            lax.fori_loop(0, (nchunk_full + NS) // NS, wbody, 0)


def _edge_pass(hs, edge_index):
    E = edge_index.shape[1]
    k = pl.kernel(
        functools.partial(_edge_body, E),
        out_type=[jax.ShapeDtypeStruct((N_NODES, N_FEAT), jnp.float32),
                  jax.ShapeDtypeStruct((N_NODES, N_FEAT), jnp.float32)],
        mesh=_mesh(),
        scratch_types=[
            pltpu.VMEM((WIN,), jnp.int32),
            pltpu.VMEM((WIN,), jnp.int32),
            pltpu.VMEM((WIN, N_FEAT), jnp.float32),
            pltpu.VMEM((16, N_FEAT), jnp.float32),
            pltpu.VMEM_SHARED((N_NODES, N_FEAT), jnp.float32),
            pltpu.SemaphoreType.DMA,
        ],
    )
    return k(hs, edge_index)


_RB = 1000


def _dinv_from_counts(c0_ref, c1_ref):
    deg = c0_ref[...] + c1_ref[...] + 1.0
    return lax.rsqrt(deg)


def _b1_body(x_ref, w_ref, c0_ref, c1_ref, o_ref):
    dinv = _dinv_from_counts(c0_ref, c1_ref)
    h = jnp.dot(x_ref[...], w_ref[...], preferred_element_type=jnp.float32,
                precision=lax.Precision.HIGHEST)
    o_ref[...] = h * dinv


def _matmul_scale(x, W, c0, c1):
    return pl.pallas_call(
        _b1_body,
        grid=(N_NODES // _RB,),
        in_specs=[
            pl.BlockSpec((_RB, N_FEAT), lambda i: (i, 0)),
            pl.BlockSpec((N_FEAT, N_FEAT), lambda i: (0, 0)),
            pl.BlockSpec((_RB, 1), lambda i: (i, 0)),
            pl.BlockSpec((_RB, 1), lambda i: (i, 0)),
        ],
        out_specs=pl.BlockSpec((_RB, N_FEAT), lambda i: (i, 0)),
        out_shape=jax.ShapeDtypeStruct((N_NODES, N_FEAT), jnp.float32),
    )(x, W, c0, c1)


def _b2_body(a0_ref, a1_ref, hs_ref, c0_ref, c1_ref, b_ref, w_ref, o_ref):
    dinv = _dinv_from_counts(c0_ref, c1_ref)
    s = a0_ref[...] + a1_ref[...] + hs_ref[...]
    o = jax.nn.relu(dinv * s + b_ref[...])
    h2 = jnp.dot(o, w_ref[...], preferred_element_type=jnp.float32,
                 precision=lax.Precision.HIGHEST)
    o_ref[...] = h2 * dinv


def _conv_finish_matmul(a0, a1, hs, c0, c1, b2d, W):
    return pl.pallas_call(
        _b2_body,
        grid=(N_NODES // _RB,),
        in_specs=[
            pl.BlockSpec((_RB, N_FEAT), lambda i: (i, 0)),
            pl.BlockSpec((_RB, N_FEAT), lambda i: (i, 0)),
            pl.BlockSpec((_RB, N_FEAT), lambda i: (i, 0)),
            pl.BlockSpec((_RB, 1), lambda i: (i, 0)),
            pl.BlockSpec((_RB, 1), lambda i: (i, 0)),
            pl.BlockSpec((1, N_FEAT), lambda i: (0, 0)),
            pl.BlockSpec((N_FEAT, N_FEAT), lambda i: (0, 0)),
        ],
        out_specs=pl.BlockSpec((_RB, N_FEAT), lambda i: (i, 0)),
        out_shape=jax.ShapeDtypeStruct((N_NODES, N_FEAT), jnp.float32),
    )(a0, a1, hs, c0, c1, b2d, W)


def _b3_body(a0_ref, a1_ref, hs_ref, c0_ref, c1_ref, b_ref, o_ref):
    dinv = _dinv_from_counts(c0_ref, c1_ref)
    s = a0_ref[...] + a1_ref[...] + hs_ref[...]
    o = jax.nn.relu(dinv * s + b_ref[...])
    part = jnp.sum(o, axis=0, keepdims=True)

    @pl.when(pl.program_id(0) == 0)
    def _():
        o_ref[...] = part

    @pl.when(pl.program_id(0) != 0)
    def _():
        o_ref[...] += part


def _conv_finish_pool(a0, a1, hs, c0, c1, b2d):
    return pl.pallas_call(
        _b3_body,
        grid=(N_NODES // _RB,),
        in_specs=[
            pl.BlockSpec((_RB, N_FEAT), lambda i: (i, 0)),
            pl.BlockSpec((_RB, N_FEAT), lambda i: (i, 0)),
            pl.BlockSpec((_RB, N_FEAT), lambda i: (i, 0)),
            pl.BlockSpec((_RB, 1), lambda i: (i, 0)),
            pl.BlockSpec((_RB, 1), lambda i: (i, 0)),
            pl.BlockSpec((1, N_FEAT), lambda i: (0, 0)),
        ],
        out_specs=pl.BlockSpec((1, N_FEAT), lambda i: (0, 0)),
        out_shape=jax.ShapeDtypeStruct((1, N_FEAT), jnp.float32),
    )(a0, a1, hs, c0, c1, b2d)


def _head_body(s_ref, w1_ref, b1_ref, w2_ref, b2_ref, o_ref):
    pooled = s_ref[...] * (1.0 / N_NODES)
    y = jax.nn.relu(
        jnp.dot(pooled, w1_ref[...], preferred_element_type=jnp.float32,
                precision=lax.Precision.HIGHEST) + b1_ref[...])
    z = jax.nn.relu(
        jnp.dot(y, w2_ref[...], preferred_element_type=jnp.float32,
                precision=lax.Precision.HIGHEST) + b2_ref[...])
    o_ref[...] = z


def _head(sums, Wl1, bl1_2d, Wl2, bl2_2d):
    return pl.pallas_call(
        _head_body,
        out_shape=jax.ShapeDtypeStruct((1, 1), jnp.float32),
    )(sums, Wl1, bl1_2d, Wl2, bl2_2d)


def kernel(x, edge_index, W1, b1, W2, b2, Wl1, bl1, Wl2, bl2):
    cnt0, cnt1 = _deg_counts(edge_index)
    c0 = cnt0.reshape(N_NODES, 1)
    c1 = cnt1.reshape(N_NODES, 1)
    hs1 = _matmul_scale(x, W1, c0, c1)
    a0, a1 = _edge_pass(hs1, edge_index)
    hs2 = _conv_finish_matmul(a0, a1, hs1, c0, c1, b1.reshape(1, -1), W2)
    a0, a1 = _edge_pass(hs2, edge_index)
    sums = _conv_finish_pool(a0, a1, hs2, c0, c1, b2.reshape(1, -1))
    out = _head(sums, Wl1, bl1.reshape(1, -1), Wl2, bl2.reshape(1, 1))
    return out


# speedup vs baseline: 32.6730x; 1.9992x over previous
import functools

import jax
import jax.numpy as jnp
from jax import lax
from jax.experimental import pallas as pl
from jax.experimental.pallas import tpu as pltpu
from jax.experimental.pallas import tpu_sc as plsc

N_NODES = 10000
N_FEAT = 128
WIN = 128
NC, NS = 2, 16
NW = NC * NS
ZCHUNK = 640
STAGE = 88

def _mesh():
    return plsc.VectorSubcoreMesh(
        core_axis_name="c", subcore_axis_name="s", num_cores=NC, num_subcores=NS
    )


def _num_windows(E):
    return E // WIN


def _worker_range(wid, nwin):
    lo = 8 * ((nwin * wid) // (NW * 8))
    hi = lax.select(wid == NW - 1, nwin, 8 * ((nwin * (wid + 1)) // (NW * 8)))
    return lo, hi


def _deg_body(nwin, dst_hbm, out0_hbm, out1_hbm, didx_v, ones_v, zb_v, cnt_sh,
              dsem):
    cid = lax.axis_index("c")
    sid = lax.axis_index("s")
    wid = cid * NS + sid

    for i in range(8):
        ones_v[pl.ds(i * 16, 16)] = jnp.full((16,), 1.0, jnp.float32)
    for i in range(ZCHUNK // 16):
        zb_v[pl.ds(i * 16, 16)] = jnp.zeros((16,), jnp.float32)

    start = sid * ZCHUNK

    @pl.when(sid < (N_NODES // ZCHUNK))
    def _():
        pltpu.sync_copy(zb_v, cnt_sh.at[pl.ds(start, ZCHUNK)])

    @pl.when(sid == (N_NODES // ZCHUNK))
    def _():
        rem = N_NODES - (N_NODES // ZCHUNK) * ZCHUNK
        pltpu.sync_copy(zb_v.at[pl.ds(0, rem)], cnt_sh.at[pl.ds(start, rem)])

    plsc.subcore_barrier()

    lo, hi = _worker_range(wid, nwin)
    cnt = hi - lo
    pltpu.sync_copy(dst_hbm.at[pl.ds(lo, STAGE)], didx_v)

    def fire(t, _):
        @pl.when(t < cnt)
        def _():
            pltpu.async_copy(ones_v, cnt_sh.at[didx_v.at[t]], dsem, add=True)

        return 0

    lax.fori_loop(0, STAGE, fire, 0)

    def drain(t, _):
        @pl.when(t < cnt)
        def _():
            pltpu.make_async_copy(ones_v, cnt_sh.at[pl.ds(0, WIN)],
                                  dsem).wait()

        return 0

    lax.fori_loop(0, STAGE, drain, 0)
    plsc.subcore_barrier()

    rem = N_NODES - (N_NODES // ZCHUNK) * ZCHUNK
    for c, out_hbm in ((0, out0_hbm), (1, out1_hbm)):
        @pl.when(cid == c)
        def _(out_hbm=out_hbm):
            @pl.when(sid < (N_NODES // ZCHUNK))
            def _():
                pltpu.sync_copy(cnt_sh.at[pl.ds(start, ZCHUNK)], zb_v)
                pltpu.sync_copy(zb_v, out_hbm.at[pl.ds(start, ZCHUNK)])

            @pl.when(sid == (N_NODES // ZCHUNK))
            def _():
                pltpu.sync_copy(cnt_sh.at[pl.ds(start, rem)],
                                zb_v.at[pl.ds(0, rem)])
                pltpu.sync_copy(zb_v.at[pl.ds(0, rem)],
                                out_hbm.at[pl.ds(start, rem)])


def _deg_counts(dst2d, nwin):
    k = pl.kernel(
        functools.partial(_deg_body, nwin),
        out_type=[jax.ShapeDtypeStruct((N_NODES,), jnp.float32),
                  jax.ShapeDtypeStruct((N_NODES,), jnp.float32)],
        mesh=_mesh(),
        scratch_types=[
            pltpu.VMEM((STAGE, WIN), jnp.int32),
            pltpu.VMEM((WIN,), jnp.float32),
            pltpu.VMEM((ZCHUNK,), jnp.float32),
            pltpu.VMEM_SHARED((N_NODES,), jnp.float32),
            pltpu.SemaphoreType.DMA,
        ],
    )
    return k(dst2d)


def _edge_body(nwin, hs_hbm, edges_hbm, out0_hbm, out1_hbm,
               sidx0_v, sidx1_v, sidx2_v, sidx3_v,
               didx0_v, didx1_v, didx2_v, didx3_v,
               rows0_v, rows1_v, zb_v, acc_sh,
               isem0, isem1, isem2, isem3, gsem0, gsem1, ssem0, ssem1):
    cid = lax.axis_index("c")
    sid = lax.axis_index("s")
    wid = cid * NS + sid

    for i in range(16):
        for j in range(8):
            zb_v[i, pl.ds(j * 16, 16)] = jnp.zeros((16,), jnp.float32)

    rstart = sid * ZCHUNK

    def zbody(j, _):
        r0 = rstart + 16 * j

        @pl.when(r0 < N_NODES)
        def _():
            pltpu.sync_copy(zb_v, acc_sh.at[pl.ds(r0, 16)])

        return 0

    lax.fori_loop(0, ZCHUNK // 16, zbody, 0)
    plsc.subcore_barrier()

    lo, hi = _worker_range(wid, nwin)
    cnt = hi - lo

    sidx = (sidx0_v, sidx1_v, sidx2_v, sidx3_v)
    didx = (didx0_v, didx1_v, didx2_v, didx3_v)
    isems = (isem0, isem1, isem2, isem3)
    rows = (rows0_v, rows1_v)
    gsems = (gsem0, gsem1)
    ssems = (ssem0, ssem1)

    def issue_idx(t, s):
        base = (lo + t) * WIN
        pltpu.async_copy(edges_hbm.at[0, pl.ds(base, WIN)], sidx[s], isems[s])
        pltpu.async_copy(edges_hbm.at[1, pl.ds(base, WIN)], didx[s], isems[s])

    def wait_idx(s):
        pltpu.make_async_copy(edges_hbm.at[0, pl.ds(0, WIN)], sidx[s],
                              isems[s]).wait()
        pltpu.make_async_copy(edges_hbm.at[0, pl.ds(0, WIN)], didx[s],
                              isems[s]).wait()

    for tt in range(4):
        @pl.when(tt < cnt)
        def _(tt=tt):
            issue_idx(tt, tt)

    def quad_body(g, _):
        for u in range(4):
            t = 4 * g + u
            b = u % 2
            s = u
            sp = (u - 1) % 4

            @pl.when(jnp.logical_and(t >= 2, t - 2 < cnt))
            def _():
                pltpu.make_async_copy(rows[b], acc_sh.at[pl.ds(0, WIN)],
                                      ssems[b]).wait()

            @pl.when(t < cnt)
            def _():
                wait_idx(s)
                pltpu.async_copy(hs_hbm.at[sidx[s]], rows[b], gsems[b])

            @pl.when(jnp.logical_and(t >= 1, t - 1 < cnt))
            def _():
                pltpu.make_async_copy(hs_hbm.at[pl.ds(0, WIN)], rows[1 - b],
                                      gsems[1 - b]).wait()
                pltpu.async_copy(rows[1 - b], acc_sh.at[didx[sp]],
                                 ssems[1 - b], add=True)

                @pl.when(t + 3 < cnt)
                def _():
                    issue_idx(t + 3, sp)

        return 0

    lax.fori_loop(0, (STAGE + 2 + 3) // 4, quad_body, 0)
    plsc.subcore_barrier()

    nchunk_full = N_NODES // WIN
    tail_rows = N_NODES - nchunk_full * WIN

    for c, out_hbm in ((0, out0_hbm), (1, out1_hbm)):
        @pl.when(cid == c)
        def _(out_hbm=out_hbm):
            def wbody(k, _):
                ci = sid + NS * k
                r0 = ci * WIN

                @pl.when(ci < nchunk_full)
                def _():
                    pltpu.sync_copy(acc_sh.at[pl.ds(r0, WIN)], rows0_v)
                    pltpu.sync_copy(rows0_v, out_hbm.at[pl.ds(r0, WIN)])

                @pl.when(ci == nchunk_full)
                def _():
                    pltpu.sync_copy(acc_sh.at[pl.ds(r0, tail_rows)],
                                    rows0_v.at[pl.ds(0, tail_rows)])
                    pltpu.sync_copy(rows0_v.at[pl.ds(0, tail_rows)],
                                    out_hbm.at[pl.ds(r0, tail_rows)])

                return 0

            lax.fori_loop(0, (nchunk_full + NS) // NS, wbody, 0)


def _edge_pass(hs, edge_index, nwin):
    k = pl.kernel(
        functools.partial(_edge_body, nwin),
        out_type=[jax.ShapeDtypeStruct((N_NODES, N_FEAT), jnp.float32),
                  jax.ShapeDtypeStruct((N_NODES, N_FEAT), jnp.float32)],
        mesh=_mesh(),
        scratch_types=(
            [pltpu.VMEM((WIN,), jnp.int32) for _ in range(8)]
            + [
                pltpu.VMEM((WIN, N_FEAT), jnp.float32),
                pltpu.VMEM((WIN, N_FEAT), jnp.float32),
                pltpu.VMEM((16, N_FEAT), jnp.float32),
                pltpu.VMEM_SHARED((N_NODES, N_FEAT), jnp.float32),
            ]
            + [pltpu.SemaphoreType.DMA for _ in range(8)]
        ),
    )
    return k(hs, edge_index)


_RB = 1000


def _dinv_from_counts(c0_ref, c1_ref):
    deg = c0_ref[...] + c1_ref[...] + 1.0
    return lax.rsqrt(deg)


def _b1_body(x_ref, w_ref, c0_ref, c1_ref, o_ref):
    dinv = _dinv_from_counts(c0_ref, c1_ref)
    h = jnp.dot(x_ref[...], w_ref[...], preferred_element_type=jnp.float32,
                precision=lax.Precision.HIGHEST)
    o_ref[...] = h * dinv


def _matmul_scale(x, W, c0, c1):
    return pl.pallas_call(
        _b1_body,
        grid=(N_NODES // _RB,),
        in_specs=[
            pl.BlockSpec((_RB, N_FEAT), lambda i: (i, 0)),
            pl.BlockSpec((N_FEAT, N_FEAT), lambda i: (0, 0)),
            pl.BlockSpec((_RB, 1), lambda i: (i, 0)),
            pl.BlockSpec((_RB, 1), lambda i: (i, 0)),
        ],
        out_specs=pl.BlockSpec((_RB, N_FEAT), lambda i: (i, 0)),
        out_shape=jax.ShapeDtypeStruct((N_NODES, N_FEAT), jnp.float32),
    )(x, W, c0, c1)


def _b2_body(a0_ref, a1_ref, hs_ref, c0_ref, c1_ref, b_ref, w_ref, o_ref):
    dinv = _dinv_from_counts(c0_ref, c1_ref)
    s = a0_ref[...] + a1_ref[...] + hs_ref[...]
    o = jax.nn.relu(dinv * s + b_ref[...])
    h2 = jnp.dot(o, w_ref[...], preferred_element_type=jnp.float32,
                 precision=lax.Precision.HIGHEST)
    o_ref[...] = h2 * dinv


def _conv_finish_matmul(a0, a1, hs, c0, c1, b2d, W):
    return pl.pallas_call(
        _b2_body,
        grid=(N_NODES // _RB,),
        in_specs=[
            pl.BlockSpec((_RB, N_FEAT), lambda i: (i, 0)),
            pl.BlockSpec((_RB, N_FEAT), lambda i: (i, 0)),
            pl.BlockSpec((_RB, N_FEAT), lambda i: (i, 0)),
            pl.BlockSpec((_RB, 1), lambda i: (i, 0)),
            pl.BlockSpec((_RB, 1), lambda i: (i, 0)),
            pl.BlockSpec((1, N_FEAT), lambda i: (0, 0)),
            pl.BlockSpec((N_FEAT, N_FEAT), lambda i: (0, 0)),
        ],
        out_specs=pl.BlockSpec((_RB, N_FEAT), lambda i: (i, 0)),
        out_shape=jax.ShapeDtypeStruct((N_NODES, N_FEAT), jnp.float32),
    )(a0, a1, hs, c0, c1, b2d, W)


def _b3_body(a0_ref, a1_ref, hs_ref, c0_ref, c1_ref, b_ref, o_ref):
    dinv = _dinv_from_counts(c0_ref, c1_ref)
    s = a0_ref[...] + a1_ref[...] + hs_ref[...]
    o = jax.nn.relu(dinv * s + b_ref[...])
    part = jnp.sum(o, axis=0, keepdims=True)

    @pl.when(pl.program_id(0) == 0)
    def _():
        o_ref[...] = part

    @pl.when(pl.program_id(0) != 0)
    def _():
        o_ref[...] += part


def _conv_finish_pool(a0, a1, hs, c0, c1, b2d):
    return pl.pallas_call(
        _b3_body,
        grid=(N_NODES // _RB,),
        in_specs=[
            pl.BlockSpec((_RB, N_FEAT), lambda i: (i, 0)),
            pl.BlockSpec((_RB, N_FEAT), lambda i: (i, 0)),
            pl.BlockSpec((_RB, N_FEAT), lambda i: (i, 0)),
            pl.BlockSpec((_RB, 1), lambda i: (i, 0)),
            pl.BlockSpec((_RB, 1), lambda i: (i, 0)),
            pl.BlockSpec((1, N_FEAT), lambda i: (0, 0)),
        ],
        out_specs=pl.BlockSpec((1, N_FEAT), lambda i: (0, 0)),
        out_shape=jax.ShapeDtypeStruct((1, N_FEAT), jnp.float32),
    )(a0, a1, hs, c0, c1, b2d)


def _head_body(s_ref, w1_ref, b1_ref, w2_ref, b2_ref, o_ref):
    pooled = s_ref[...] * (1.0 / N_NODES)
    y = jax.nn.relu(
        jnp.dot(pooled, w1_ref[...], preferred_element_type=jnp.float32,
                precision=lax.Precision.HIGHEST) + b1_ref[...])
    z = jax.nn.relu(
        jnp.dot(y, w2_ref[...], preferred_element_type=jnp.float32,
                precision=lax.Precision.HIGHEST) + b2_ref[...])
    o_ref[...] = z


def _head(sums, Wl1, bl1_2d, Wl2, bl2_2d):
    return pl.pallas_call(
        _head_body,
        out_shape=jax.ShapeDtypeStruct((1, 1), jnp.float32),
    )(sums, Wl1, bl1_2d, Wl2, bl2_2d)


def kernel(x, edge_index, W1, b1, W2, b2, Wl1, bl1, Wl2, bl2):
    nwin = edge_index.shape[1] // WIN
    pad = (-nwin) % 8
    dst2d = jnp.pad(edge_index[1].reshape(-1, WIN), ((0, pad), (0, 0)))
    cnt0, cnt1 = _deg_counts(dst2d, nwin)
    c0 = cnt0.reshape(N_NODES, 1)
    c1 = cnt1.reshape(N_NODES, 1)
    hs1 = _matmul_scale(x, W1, c0, c1)
    a0, a1 = _edge_pass(hs1, edge_index, nwin)
    hs2 = _conv_finish_matmul(a0, a1, hs1, c0, c1, b1.reshape(1, -1), W2)
    a0, a1 = _edge_pass(hs2, edge_index, nwin)
    sums = _conv_finish_pool(a0, a1, hs2, c0, c1, b2.reshape(1, -1))
    out = _head(sums, Wl1, bl1.reshape(1, -1), Wl2, bl2.reshape(1, 1))
    return out


# speedup vs baseline: 34.3171x; 1.0503x over previous
import functools

import jax
import jax.numpy as jnp
from jax import lax
from jax.experimental import pallas as pl
from jax.experimental.pallas import tpu as pltpu
from jax.experimental.pallas import tpu_sc as plsc

N_NODES = 10000
N_FEAT = 128
WIN = 128
NC, NS = 2, 16
NW = NC * NS
ZCHUNK = 640
STAGE = 88
NBUF = 3
IBUF = 6

def _mesh():
    return plsc.VectorSubcoreMesh(
        core_axis_name="c", subcore_axis_name="s", num_cores=NC, num_subcores=NS
    )


def _num_windows(E):
    return E // WIN


def _worker_range(wid, nwin):
    lo = 8 * ((nwin * wid) // (NW * 8))
    hi = lax.select(wid == NW - 1, nwin, 8 * ((nwin * (wid + 1)) // (NW * 8)))
    return lo, hi


def _deg_body(nwin, dst_hbm, out0_hbm, out1_hbm, didx_v, ones_v, zb_v, cnt_sh,
              dsem):
    cid = lax.axis_index("c")
    sid = lax.axis_index("s")
    wid = cid * NS + sid

    for i in range(WIN // 16):
        ones_v[pl.ds(i * 16, 16)] = jnp.full((16,), 1.0, jnp.float32)
    for i in range(ZCHUNK // 16):
        zb_v[pl.ds(i * 16, 16)] = jnp.zeros((16,), jnp.float32)

    start = sid * ZCHUNK

    @pl.when(sid < (N_NODES // ZCHUNK))
    def _():
        pltpu.sync_copy(zb_v, cnt_sh.at[pl.ds(start, ZCHUNK)])

    @pl.when(sid == (N_NODES // ZCHUNK))
    def _():
        rem = N_NODES - (N_NODES // ZCHUNK) * ZCHUNK
        pltpu.sync_copy(zb_v.at[pl.ds(0, rem)], cnt_sh.at[pl.ds(start, rem)])

    plsc.subcore_barrier()

    lo, hi = _worker_range(wid, nwin)
    cnt = hi - lo
    pltpu.sync_copy(dst_hbm.at[pl.ds(lo, STAGE)], didx_v)

    def fire(t, _):
        @pl.when(t < cnt)
        def _():
            pltpu.async_copy(ones_v, cnt_sh.at[didx_v.at[t]], dsem, add=True)

        return 0

    lax.fori_loop(0, STAGE, fire, 0)

    def drain(t, _):
        @pl.when(t < cnt)
        def _():
            pltpu.make_async_copy(ones_v, cnt_sh.at[pl.ds(0, WIN)],
                                  dsem).wait()

        return 0

    lax.fori_loop(0, STAGE, drain, 0)
    plsc.subcore_barrier()

    rem = N_NODES - (N_NODES // ZCHUNK) * ZCHUNK
    for c, out_hbm in ((0, out0_hbm), (1, out1_hbm)):
        @pl.when(cid == c)
        def _(out_hbm=out_hbm):
            @pl.when(sid < (N_NODES // ZCHUNK))
            def _():
                pltpu.sync_copy(cnt_sh.at[pl.ds(start, ZCHUNK)], zb_v)
                pltpu.sync_copy(zb_v, out_hbm.at[pl.ds(start, ZCHUNK)])

            @pl.when(sid == (N_NODES // ZCHUNK))
            def _():
                pltpu.sync_copy(cnt_sh.at[pl.ds(start, rem)],
                                zb_v.at[pl.ds(0, rem)])
                pltpu.sync_copy(zb_v.at[pl.ds(0, rem)],
                                out_hbm.at[pl.ds(start, rem)])


def _deg_counts(dst2d, nwin):
    k = pl.kernel(
        functools.partial(_deg_body, nwin),
        out_type=[jax.ShapeDtypeStruct((N_NODES,), jnp.float32),
                  jax.ShapeDtypeStruct((N_NODES,), jnp.float32)],
        mesh=_mesh(),
        scratch_types=[
            pltpu.VMEM((STAGE, WIN), jnp.int32),
            pltpu.VMEM((WIN,), jnp.float32),
            pltpu.VMEM((ZCHUNK,), jnp.float32),
            pltpu.VMEM_SHARED((N_NODES,), jnp.float32),
            pltpu.SemaphoreType.DMA,
        ],
    )
    return k(dst2d)


def _edge_body(nwin, hs_hbm, edges_hbm, out0_hbm, out1_hbm, *scr):
    sidx = scr[0:IBUF]
    didx = scr[IBUF:2 * IBUF]
    rows = scr[2 * IBUF:2 * IBUF + NBUF]
    acc_sh = scr[2 * IBUF + NBUF]
    isems = scr[2 * IBUF + NBUF + 1:3 * IBUF + NBUF + 1]
    gsems = scr[3 * IBUF + NBUF + 1:3 * IBUF + 2 * NBUF + 1]
    ssems = scr[3 * IBUF + 2 * NBUF + 1:3 * IBUF + 3 * NBUF + 1]

    cid = lax.axis_index("c")
    sid = lax.axis_index("s")
    wid = cid * NS + sid

    def zfill(r, _):
        for j in range(N_FEAT // 16):
            rows[0][r, pl.ds(j * 16, 16)] = jnp.zeros((16,), jnp.float32)
        return 0

    lax.fori_loop(0, WIN, zfill, 0)

    nchunk = N_NODES // WIN
    tail_rows = N_NODES - nchunk * WIN

    def zbody(k, _):
        ci = sid + NS * k
        r0 = ci * WIN

        @pl.when(ci < nchunk)
        def _():
            pltpu.sync_copy(rows[0], acc_sh.at[pl.ds(r0, WIN)])

        @pl.when(ci == nchunk)
        def _():
            pltpu.sync_copy(rows[0].at[pl.ds(0, tail_rows)],
                            acc_sh.at[pl.ds(r0, tail_rows)])

        return 0

    lax.fori_loop(0, (nchunk + NS) // NS, zbody, 0)

    lo = (nwin * wid) // NW
    hi = (nwin * (wid + 1)) // NW
    cnt = hi - lo

    def issue_idx(t, s):
        base = (lo + t) * WIN
        pltpu.async_copy(edges_hbm.at[0, pl.ds(base, WIN)], sidx[s], isems[s])
        pltpu.async_copy(edges_hbm.at[1, pl.ds(base, WIN)], didx[s], isems[s])

    def wait_idx(s):
        pltpu.make_async_copy(edges_hbm.at[0, pl.ds(0, WIN)], sidx[s],
                              isems[s]).wait()
        pltpu.make_async_copy(edges_hbm.at[0, pl.ds(0, WIN)], didx[s],
                              isems[s]).wait()

    for tt in range(IBUF):
        @pl.when(tt < cnt)
        def _(tt=tt):
            issue_idx(tt, tt)

    plsc.subcore_barrier()

    wait_idx(0)
    pltpu.async_copy(hs_hbm.at[sidx[0]], rows[0], gsems[0])

    def hex_body(g, _):
        for u in range(IBUF):
            t = 6 * g + u
            rm1 = (u + 2) % NBUF
            rm2 = (u + 1) % NBUF
            im1 = (u + 5) % IBUF
            ip1 = (u + 1) % IBUF

            @pl.when(jnp.logical_and(t >= 1, t - 1 < cnt))
            def _():
                pltpu.make_async_copy(hs_hbm.at[pl.ds(0, WIN)], rows[rm1],
                                      gsems[rm1]).wait()
                pltpu.async_copy(rows[rm1], acc_sh.at[didx[im1]],
                                 ssems[rm1], add=True)

                @pl.when(t + 5 < cnt)
                def _():
                    issue_idx(t + 5, im1)

            @pl.when(jnp.logical_and(t >= 2, t - 2 < cnt))
            def _():
                pltpu.make_async_copy(rows[rm2], acc_sh.at[pl.ds(0, WIN)],
                                      ssems[rm2]).wait()

            @pl.when(t + 1 < cnt)
            def _():
                wait_idx(ip1)
                pltpu.async_copy(hs_hbm.at[sidx[ip1]], rows[rm2], gsems[rm2])

        return 0

    lax.fori_loop(0, (STAGE + 2 + 5) // 6, hex_body, 0)
    plsc.subcore_barrier()

    for c, out_hbm in ((0, out0_hbm), (1, out1_hbm)):
        @pl.when(cid == c)
        def _(out_hbm=out_hbm):
            def wbody(k, _):
                ci = sid + NS * k
                r0 = ci * WIN

                @pl.when(ci < nchunk)
                def _():
                    pltpu.sync_copy(acc_sh.at[pl.ds(r0, WIN)], rows[0])
                    pltpu.sync_copy(rows[0], out_hbm.at[pl.ds(r0, WIN)])

                @pl.when(ci == nchunk)
                def _():
                    pltpu.sync_copy(acc_sh.at[pl.ds(r0, tail_rows)],
                                    rows[1].at[pl.ds(0, tail_rows)])
                    pltpu.sync_copy(rows[1].at[pl.ds(0, tail_rows)],
                                    out_hbm.at[pl.ds(r0, tail_rows)])

                return 0

            lax.fori_loop(0, (nchunk + NS) // NS, wbody, 0)


def _edge_pass(hs, edge_index, nwin):
    k = pl.kernel(
        functools.partial(_edge_body, nwin),
        out_type=[jax.ShapeDtypeStruct((N_NODES, N_FEAT), jnp.float32),
                  jax.ShapeDtypeStruct((N_NODES, N_FEAT), jnp.float32)],
        mesh=_mesh(),
        scratch_types=(
            [pltpu.VMEM((WIN,), jnp.int32) for _ in range(2 * IBUF)]
            + [pltpu.VMEM((WIN, N_FEAT), jnp.float32) for _ in range(NBUF)]
            + [pltpu.VMEM_SHARED((N_NODES, N_FEAT), jnp.float32)]
            + [pltpu.SemaphoreType.DMA for _ in range(IBUF + 2 * NBUF)]
        ),
    )
    return k(hs, edge_index)


_RB = 1000


def _dinv_from_counts(c0_ref, c1_ref):
    deg = c0_ref[...] + c1_ref[...] + 1.0
    return lax.rsqrt(deg)


def _b1_body(x_ref, w_ref, c0_ref, c1_ref, o_ref):
    dinv = _dinv_from_counts(c0_ref, c1_ref)
    h = jnp.dot(x_ref[...], w_ref[...], preferred_element_type=jnp.float32,
                precision=lax.Precision.HIGHEST)
    o_ref[...] = h * dinv


def _matmul_scale(x, W, c0, c1):
    return pl.pallas_call(
        _b1_body,
        grid=(N_NODES // _RB,),
        in_specs=[
            pl.BlockSpec((_RB, N_FEAT), lambda i: (i, 0)),
            pl.BlockSpec((N_FEAT, N_FEAT), lambda i: (0, 0)),
            pl.BlockSpec((_RB, 1), lambda i: (i, 0)),
            pl.BlockSpec((_RB, 1), lambda i: (i, 0)),
        ],
        out_specs=pl.BlockSpec((_RB, N_FEAT), lambda i: (i, 0)),
        out_shape=jax.ShapeDtypeStruct((N_NODES, N_FEAT), jnp.float32),
    )(x, W, c0, c1)


def _b2_body(a0_ref, a1_ref, hs_ref, c0_ref, c1_ref, b_ref, w_ref, o_ref):
    dinv = _dinv_from_counts(c0_ref, c1_ref)
    s = a0_ref[...] + a1_ref[...] + hs_ref[...]
    o = jax.nn.relu(dinv * s + b_ref[...])
    h2 = jnp.dot(o, w_ref[...], preferred_element_type=jnp.float32,
                 precision=lax.Precision.HIGHEST)
    o_ref[...] = h2 * dinv


def _conv_finish_matmul(a0, a1, hs, c0, c1, b2d, W):
    return pl.pallas_call(
        _b2_body,
        grid=(N_NODES // _RB,),
        in_specs=[
            pl.BlockSpec((_RB, N_FEAT), lambda i: (i, 0)),
            pl.BlockSpec((_RB, N_FEAT), lambda i: (i, 0)),
            pl.BlockSpec((_RB, N_FEAT), lambda i: (i, 0)),
            pl.BlockSpec((_RB, 1), lambda i: (i, 0)),
            pl.BlockSpec((_RB, 1), lambda i: (i, 0)),
            pl.BlockSpec((1, N_FEAT), lambda i: (0, 0)),
            pl.BlockSpec((N_FEAT, N_FEAT), lambda i: (0, 0)),
        ],
        out_specs=pl.BlockSpec((_RB, N_FEAT), lambda i: (i, 0)),
        out_shape=jax.ShapeDtypeStruct((N_NODES, N_FEAT), jnp.float32),
    )(a0, a1, hs, c0, c1, b2d, W)


def _b3_body(a0_ref, a1_ref, hs_ref, c0_ref, c1_ref, b_ref, o_ref):
    dinv = _dinv_from_counts(c0_ref, c1_ref)
    s = a0_ref[...] + a1_ref[...] + hs_ref[...]
    o = jax.nn.relu(dinv * s + b_ref[...])
    part = jnp.sum(o, axis=0, keepdims=True)

    @pl.when(pl.program_id(0) == 0)
    def _():
        o_ref[...] = part

    @pl.when(pl.program_id(0) != 0)
    def _():
        o_ref[...] += part


def _conv_finish_pool(a0, a1, hs, c0, c1, b2d):
    return pl.pallas_call(
        _b3_body,
        grid=(N_NODES // _RB,),
        in_specs=[
            pl.BlockSpec((_RB, N_FEAT), lambda i: (i, 0)),
            pl.BlockSpec((_RB, N_FEAT), lambda i: (i, 0)),
            pl.BlockSpec((_RB, N_FEAT), lambda i: (i, 0)),
            pl.BlockSpec((_RB, 1), lambda i: (i, 0)),
            pl.BlockSpec((_RB, 1), lambda i: (i, 0)),
            pl.BlockSpec((1, N_FEAT), lambda i: (0, 0)),
        ],
        out_specs=pl.BlockSpec((1, N_FEAT), lambda i: (0, 0)),
        out_shape=jax.ShapeDtypeStruct((1, N_FEAT), jnp.float32),
    )(a0, a1, hs, c0, c1, b2d)


def _head_body(s_ref, w1_ref, b1_ref, w2_ref, b2_ref, o_ref):
    pooled = s_ref[...] * (1.0 / N_NODES)
    y = jax.nn.relu(
        jnp.dot(pooled, w1_ref[...], preferred_element_type=jnp.float32,
                precision=lax.Precision.HIGHEST) + b1_ref[...])
    z = jax.nn.relu(
        jnp.dot(y, w2_ref[...], preferred_element_type=jnp.float32,
                precision=lax.Precision.HIGHEST) + b2_ref[...])
    o_ref[...] = z


def _head(sums, Wl1, bl1_2d, Wl2, bl2_2d):
    return pl.pallas_call(
        _head_body,
        out_shape=jax.ShapeDtypeStruct((1, 1), jnp.float32),
    )(sums, Wl1, bl1_2d, Wl2, bl2_2d)


def kernel(x, edge_index, W1, b1, W2, b2, Wl1, bl1, Wl2, bl2):
    nwin = edge_index.shape[1] // WIN
    pad = (-nwin) % 8
    dst2d = jnp.pad(edge_index[1].reshape(-1, WIN), ((0, pad), (0, 0)))
    cnt0, cnt1 = _deg_counts(dst2d, nwin)
    c0 = cnt0.reshape(N_NODES, 1)
    c1 = cnt1.reshape(N_NODES, 1)
    hs1 = _matmul_scale(x, W1, c0, c1)
    a0, a1 = _edge_pass(hs1, edge_index, nwin)
    hs2 = _conv_finish_matmul(a0, a1, hs1, c0, c1, b1.reshape(1, -1), W2)
    a0, a1 = _edge_pass(hs2, edge_index, nwin)
    sums = _conv_finish_pool(a0, a1, hs2, c0, c1, b2.reshape(1, -1))
    out = _head(sums, Wl1, bl1.reshape(1, -1), Wl2, bl2.reshape(1, 1))
    return out


# speedup vs baseline: 35.5332x; 1.0354x over previous
import functools

import jax
import jax.numpy as jnp
from jax import lax
from jax.experimental import pallas as pl
from jax.experimental.pallas import tpu as pltpu
from jax.experimental.pallas import tpu_sc as plsc

N_NODES = 10000
N_FEAT = 128
WIN = 128
NC, NS = 2, 16
NW = NC * NS
ZCHUNK = 640
STAGE = 88
NBUF = 3
IBUF = 6

def _mesh():
    return plsc.VectorSubcoreMesh(
        core_axis_name="c", subcore_axis_name="s", num_cores=NC, num_subcores=NS
    )


def _num_windows(E):
    return E // WIN


def _worker_range(wid, nwin):
    lo = 8 * ((nwin * wid) // (NW * 8))
    hi = lax.select(wid == NW - 1, nwin, 8 * ((nwin * (wid + 1)) // (NW * 8)))
    return lo, hi


def _deg_body(nwin, dst_hbm, out0_hbm, out1_hbm, didx_v, ones_v, zb_v, cnt_sh,
              dsem):
    cid = lax.axis_index("c")
    sid = lax.axis_index("s")
    wid = cid * NS + sid

    for i in range(WIN // 16):
        ones_v[pl.ds(i * 16, 16)] = jnp.full((16,), 1.0, jnp.float32)
    for i in range(ZCHUNK // 16):
        zb_v[pl.ds(i * 16, 16)] = jnp.zeros((16,), jnp.float32)

    start = sid * ZCHUNK

    @pl.when(sid < (N_NODES // ZCHUNK))
    def _():
        pltpu.sync_copy(zb_v, cnt_sh.at[pl.ds(start, ZCHUNK)])

    @pl.when(sid == (N_NODES // ZCHUNK))
    def _():
        rem = N_NODES - (N_NODES // ZCHUNK) * ZCHUNK
        pltpu.sync_copy(zb_v.at[pl.ds(0, rem)], cnt_sh.at[pl.ds(start, rem)])

    plsc.subcore_barrier()

    lo, hi = _worker_range(wid, nwin)
    cnt = hi - lo
    pltpu.sync_copy(dst_hbm.at[pl.ds(lo, STAGE)], didx_v)

    def fire(t, _):
        @pl.when(t < cnt)
        def _():
            pltpu.async_copy(ones_v, cnt_sh.at[didx_v.at[t]], dsem, add=True)

        return 0

    lax.fori_loop(0, STAGE, fire, 0)

    def drain(t, _):
        @pl.when(t < cnt)
        def _():
            pltpu.make_async_copy(ones_v, cnt_sh.at[pl.ds(0, WIN)],
                                  dsem).wait()

        return 0

    lax.fori_loop(0, STAGE, drain, 0)
    plsc.subcore_barrier()

    rem = N_NODES - (N_NODES // ZCHUNK) * ZCHUNK
    for c, out_hbm in ((0, out0_hbm), (1, out1_hbm)):
        @pl.when(cid == c)
        def _(out_hbm=out_hbm):
            @pl.when(sid < (N_NODES // ZCHUNK))
            def _():
                pltpu.sync_copy(cnt_sh.at[pl.ds(start, ZCHUNK)], zb_v)
                pltpu.sync_copy(zb_v, out_hbm.at[pl.ds(start, ZCHUNK)])

            @pl.when(sid == (N_NODES // ZCHUNK))
            def _():
                pltpu.sync_copy(cnt_sh.at[pl.ds(start, rem)],
                                zb_v.at[pl.ds(0, rem)])
                pltpu.sync_copy(zb_v.at[pl.ds(0, rem)],
                                out_hbm.at[pl.ds(start, rem)])


def _deg_counts(dst2d, nwin):
    k = pl.kernel(
        functools.partial(_deg_body, nwin),
        out_type=[jax.ShapeDtypeStruct((N_NODES,), jnp.float32),
                  jax.ShapeDtypeStruct((N_NODES,), jnp.float32)],
        mesh=_mesh(),
        scratch_types=[
            pltpu.VMEM((STAGE, WIN), jnp.int32),
            pltpu.VMEM((WIN,), jnp.float32),
            pltpu.VMEM((ZCHUNK,), jnp.float32),
            pltpu.VMEM_SHARED((N_NODES,), jnp.float32),
            pltpu.SemaphoreType.DMA,
        ],
    )
    return k(dst2d)


def _edge_body(nwin, hs_hbm, edges_hbm, out0_hbm, out1_hbm, *scr):
    sidx = scr[0:IBUF]
    didx = scr[IBUF:2 * IBUF]
    rows = scr[2 * IBUF:2 * IBUF + NBUF]
    acc_sh = scr[2 * IBUF + NBUF]
    isems = scr[2 * IBUF + NBUF + 1:3 * IBUF + NBUF + 1]
    gsems = scr[3 * IBUF + NBUF + 1:3 * IBUF + 2 * NBUF + 1]
    ssems = scr[3 * IBUF + 2 * NBUF + 1:3 * IBUF + 3 * NBUF + 1]

    cid = lax.axis_index("c")
    sid = lax.axis_index("s")
    wid = cid * NS + sid

    def zfill(r, _):
        for j in range(N_FEAT // 16):
            rows[0][r, pl.ds(j * 16, 16)] = jnp.zeros((16,), jnp.float32)
        return 0

    lax.fori_loop(0, WIN, zfill, 0)

    nchunk = N_NODES // WIN
    tail_rows = N_NODES - nchunk * WIN

    def zbody(k, _):
        ci = sid + NS * k
        r0 = ci * WIN

        @pl.when(ci < nchunk)
        def _():
            pltpu.sync_copy(rows[0], acc_sh.at[pl.ds(r0, WIN)])

        @pl.when(ci == nchunk)
        def _():
            pltpu.sync_copy(rows[0].at[pl.ds(0, tail_rows)],
                            acc_sh.at[pl.ds(r0, tail_rows)])

        return 0

    lax.fori_loop(0, (nchunk + NS) // NS, zbody, 0)

    lo = (nwin * wid) // NW
    hi = (nwin * (wid + 1)) // NW
    cnt = hi - lo

    def issue_idx(t, s):
        base = (lo + t) * WIN
        pltpu.async_copy(edges_hbm.at[0, pl.ds(base, WIN)], sidx[s], isems[s])
        pltpu.async_copy(edges_hbm.at[1, pl.ds(base, WIN)], didx[s], isems[s])

    def wait_idx(s):
        pltpu.make_async_copy(edges_hbm.at[0, pl.ds(0, WIN)], sidx[s],
                              isems[s]).wait()
        pltpu.make_async_copy(edges_hbm.at[0, pl.ds(0, WIN)], didx[s],
                              isems[s]).wait()

    for tt in range(IBUF):
        @pl.when(tt < cnt)
        def _(tt=tt):
            issue_idx(tt, tt)

    plsc.subcore_barrier()

    wait_idx(0)
    pltpu.async_copy(hs_hbm.at[sidx[0]], rows[0], gsems[0])

    def hex_body(g, _):
        for u in range(IBUF):
            t = 6 * g + u
            rm1 = (u + 2) % NBUF
            rm2 = (u + 1) % NBUF
            im1 = (u + 5) % IBUF
            ip1 = (u + 1) % IBUF

            @pl.when(jnp.logical_and(t >= 1, t - 1 < cnt))
            def _():
                pltpu.make_async_copy(hs_hbm.at[pl.ds(0, WIN)], rows[rm1],
                                      gsems[rm1]).wait()
                pltpu.async_copy(rows[rm1], acc_sh.at[didx[im1]],
                                 ssems[rm1], add=True)

                @pl.when(t + 5 < cnt)
                def _():
                    issue_idx(t + 5, im1)

            @pl.when(jnp.logical_and(t >= 2, t - 2 < cnt))
            def _():
                pltpu.make_async_copy(rows[rm2], acc_sh.at[pl.ds(0, WIN)],
                                      ssems[rm2]).wait()

            @pl.when(t + 1 < cnt)
            def _():
                wait_idx(ip1)
                pltpu.async_copy(hs_hbm.at[sidx[ip1]], rows[rm2], gsems[rm2])

        return 0

    lax.fori_loop(0, (STAGE + 2 + 5) // 6, hex_body, 0)
    plsc.subcore_barrier()

    for c, out_hbm in ((0, out0_hbm), (1, out1_hbm)):
        @pl.when(cid == c)
        def _(out_hbm=out_hbm):
            def wbody(k, _):
                ci = sid + NS * k
                r0 = ci * WIN

                @pl.when(ci < nchunk)
                def _():
                    pltpu.sync_copy(acc_sh.at[pl.ds(r0, WIN)], rows[0])
                    pltpu.sync_copy(rows[0], out_hbm.at[pl.ds(r0, WIN)])

                @pl.when(ci == nchunk)
                def _():
                    pltpu.sync_copy(acc_sh.at[pl.ds(r0, tail_rows)],
                                    rows[1].at[pl.ds(0, tail_rows)])
                    pltpu.sync_copy(rows[1].at[pl.ds(0, tail_rows)],
                                    out_hbm.at[pl.ds(r0, tail_rows)])

                return 0

            lax.fori_loop(0, (nchunk + NS) // NS, wbody, 0)


def _edge_pass(hs, edge_index, nwin):
    k = pl.kernel(
        functools.partial(_edge_body, nwin),
        out_type=[jax.ShapeDtypeStruct((N_NODES, N_FEAT), jnp.float32),
                  jax.ShapeDtypeStruct((N_NODES, N_FEAT), jnp.float32)],
        mesh=_mesh(),
        scratch_types=(
            [pltpu.VMEM((WIN,), jnp.int32) for _ in range(2 * IBUF)]
            + [pltpu.VMEM((WIN, N_FEAT), jnp.float32) for _ in range(NBUF)]
            + [pltpu.VMEM_SHARED((N_NODES, N_FEAT), jnp.float32)]
            + [pltpu.SemaphoreType.DMA for _ in range(IBUF + 2 * NBUF)]
        ),
    )
    return k(hs, edge_index)


_RB = 2000


def _dinv_from_counts(c0_ref, c1_ref):
    deg = c0_ref[...] + c1_ref[...] + 1.0
    return lax.rsqrt(deg)


def _b1_body(x_ref, w_ref, c0_ref, c1_ref, o_ref):
    dinv = _dinv_from_counts(c0_ref, c1_ref)
    h = jnp.dot(x_ref[...], w_ref[...], preferred_element_type=jnp.float32,
                precision=lax.Precision.HIGHEST)
    o_ref[...] = h * dinv


def _matmul_scale(x, W, c0, c1):
    return pl.pallas_call(
        _b1_body,
        grid=(N_NODES // _RB,),
        in_specs=[
            pl.BlockSpec((_RB, N_FEAT), lambda i: (i, 0)),
            pl.BlockSpec((N_FEAT, N_FEAT), lambda i: (0, 0)),
            pl.BlockSpec((_RB, 1), lambda i: (i, 0)),
            pl.BlockSpec((_RB, 1), lambda i: (i, 0)),
        ],
        out_specs=pl.BlockSpec((_RB, N_FEAT), lambda i: (i, 0)),
        out_shape=jax.ShapeDtypeStruct((N_NODES, N_FEAT), jnp.float32),
    )(x, W, c0, c1)


def _b2_body(a0_ref, a1_ref, hs_ref, c0_ref, c1_ref, b_ref, w_ref, o_ref):
    dinv = _dinv_from_counts(c0_ref, c1_ref)
    s = a0_ref[...] + a1_ref[...] + hs_ref[...]
    o = jax.nn.relu(dinv * s + b_ref[...])
    h2 = jnp.dot(o, w_ref[...], preferred_element_type=jnp.float32,
                 precision=lax.Precision.HIGHEST)
    o_ref[...] = h2 * dinv


def _conv_finish_matmul(a0, a1, hs, c0, c1, b2d, W):
    return pl.pallas_call(
        _b2_body,
        grid=(N_NODES // _RB,),
        in_specs=[
            pl.BlockSpec((_RB, N_FEAT), lambda i: (i, 0)),
            pl.BlockSpec((_RB, N_FEAT), lambda i: (i, 0)),
            pl.BlockSpec((_RB, N_FEAT), lambda i: (i, 0)),
            pl.BlockSpec((_RB, 1), lambda i: (i, 0)),
            pl.BlockSpec((_RB, 1), lambda i: (i, 0)),
            pl.BlockSpec((1, N_FEAT), lambda i: (0, 0)),
            pl.BlockSpec((N_FEAT, N_FEAT), lambda i: (0, 0)),
        ],
        out_specs=pl.BlockSpec((_RB, N_FEAT), lambda i: (i, 0)),
        out_shape=jax.ShapeDtypeStruct((N_NODES, N_FEAT), jnp.float32),
    )(a0, a1, hs, c0, c1, b2d, W)


def _b3_body(a0_ref, a1_ref, hs_ref, c0_ref, c1_ref, b_ref, w1_ref, bl1_ref,
             w2_ref, bl2_ref, o_ref, sums_ref):
    dinv = _dinv_from_counts(c0_ref, c1_ref)
    s = a0_ref[...] + a1_ref[...] + hs_ref[...]
    o = jax.nn.relu(dinv * s + b_ref[...])
    part = jnp.sum(o, axis=0, keepdims=True)

    @pl.when(pl.program_id(0) == 0)
    def _():
        sums_ref[...] = part

    @pl.when(pl.program_id(0) != 0)
    def _():
        sums_ref[...] += part

    @pl.when(pl.program_id(0) == pl.num_programs(0) - 1)
    def _():
        pooled = sums_ref[...] * (1.0 / N_NODES)
        y = jax.nn.relu(
            jnp.dot(pooled, w1_ref[...], preferred_element_type=jnp.float32,
                    precision=lax.Precision.HIGHEST) + bl1_ref[...])
        z = jax.nn.relu(
            jnp.dot(y, w2_ref[...], preferred_element_type=jnp.float32,
                    precision=lax.Precision.HIGHEST) + bl2_ref[...])
        o_ref[...] = z


def _conv_finish_pool_head(a0, a1, hs, c0, c1, b2d, Wl1, bl1_2d, Wl2, bl2_2d):
    return pl.pallas_call(
        _b3_body,
        grid=(N_NODES // _RB,),
        in_specs=[
            pl.BlockSpec((_RB, N_FEAT), lambda i: (i, 0)),
            pl.BlockSpec((_RB, N_FEAT), lambda i: (i, 0)),
            pl.BlockSpec((_RB, N_FEAT), lambda i: (i, 0)),
            pl.BlockSpec((_RB, 1), lambda i: (i, 0)),
            pl.BlockSpec((_RB, 1), lambda i: (i, 0)),
            pl.BlockSpec((1, N_FEAT), lambda i: (0, 0)),
            pl.BlockSpec((N_FEAT, N_FEAT), lambda i: (0, 0)),
            pl.BlockSpec((1, N_FEAT), lambda i: (0, 0)),
            pl.BlockSpec((N_FEAT, 1), lambda i: (0, 0)),
            pl.BlockSpec((1, 1), lambda i: (0, 0)),
        ],
        out_specs=pl.BlockSpec((1, 1), lambda i: (0, 0)),
        out_shape=jax.ShapeDtypeStruct((1, 1), jnp.float32),
        scratch_shapes=[pltpu.VMEM((1, N_FEAT), jnp.float32)],
    )(a0, a1, hs, c0, c1, b2d, Wl1, bl1_2d, Wl2, bl2_2d)


def kernel(x, edge_index, W1, b1, W2, b2, Wl1, bl1, Wl2, bl2):
    nwin = edge_index.shape[1] // WIN
    pad = (-nwin) % 8
    dst2d = jnp.pad(edge_index[1].reshape(-1, WIN), ((0, pad), (0, 0)))
    cnt0, cnt1 = _deg_counts(dst2d, nwin)
    c0 = cnt0.reshape(N_NODES, 1)
    c1 = cnt1.reshape(N_NODES, 1)
    hs1 = _matmul_scale(x, W1, c0, c1)
    a0, a1 = _edge_pass(hs1, edge_index, nwin)
    hs2 = _conv_finish_matmul(a0, a1, hs1, c0, c1, b1.reshape(1, -1), W2)
    a0, a1 = _edge_pass(hs2, edge_index, nwin)
    out = _conv_finish_pool_head(a0, a1, hs2, c0, c1, b2.reshape(1, -1),
                                 Wl1, bl1.reshape(1, -1), Wl2,
                                 bl2.reshape(1, 1))
    return out


# speedup vs baseline: 36.3415x; 1.0227x over previous
import functools

import jax
import jax.numpy as jnp
from jax import lax
from jax.experimental import pallas as pl
from jax.experimental.pallas import tpu as pltpu
from jax.experimental.pallas import tpu_sc as plsc

N_NODES = 10000
N_FEAT = 128
WIN = 128
NC, NS = 2, 16
NW = NC * NS
ZCHUNK = 640
NBUF = 3
IBUF = 6

def _mesh():
    return plsc.VectorSubcoreMesh(
        core_axis_name="c", subcore_axis_name="s", num_cores=NC, num_subcores=NS
    )


def _deg_body(nwin, edges_hbm, out0_hbm, out1_hbm, *scr):
    DSLOT = 8
    didx = scr[0:DSLOT]
    ones_v = scr[DSLOT]
    zb_v = scr[DSLOT + 1]
    cnt_sh = scr[DSLOT + 2]
    isems = scr[DSLOT + 3:2 * DSLOT + 3]
    ssems = scr[2 * DSLOT + 3:3 * DSLOT + 3]

    cid = lax.axis_index("c")
    sid = lax.axis_index("s")
    wid = cid * NS + sid

    for i in range(WIN // 16):
        ones_v[pl.ds(i * 16, 16)] = jnp.full((16,), 1.0, jnp.float32)
    for i in range(ZCHUNK // 16):
        zb_v[pl.ds(i * 16, 16)] = jnp.zeros((16,), jnp.float32)

    start = sid * ZCHUNK

    @pl.when(sid < (N_NODES // ZCHUNK))
    def _():
        pltpu.sync_copy(zb_v, cnt_sh.at[pl.ds(start, ZCHUNK)])

    @pl.when(sid == (N_NODES // ZCHUNK))
    def _():
        rem = N_NODES - (N_NODES // ZCHUNK) * ZCHUNK
        pltpu.sync_copy(zb_v.at[pl.ds(0, rem)], cnt_sh.at[pl.ds(start, rem)])

    lo = (nwin * wid) // NW
    hi = (nwin * (wid + 1)) // NW
    cnt = hi - lo

    def issue_idx(t, s):
        pltpu.async_copy(edges_hbm.at[1, pl.ds((lo + t) * WIN, WIN)],
                         didx[s], isems[s])

    for ss in range(DSLOT):
        @pl.when(ss < cnt)
        def _(ss=ss):
            issue_idx(ss, ss)

    plsc.subcore_barrier()

    maxw = (nwin + NW - 1) // NW

    def ring_body(g, _):
        for u in range(DSLOT):
            t = DSLOT * g + u
            sp = (u + 4) % DSLOT

            @pl.when(t < cnt)
            def _():
                pltpu.make_async_copy(edges_hbm.at[1, pl.ds(0, WIN)],
                                      didx[u], isems[u]).wait()
                pltpu.async_copy(ones_v, cnt_sh.at[didx[u]], ssems[u],
                                 add=True)

            @pl.when(jnp.logical_and(t >= 4, t - 4 < cnt))
            def _():
                pltpu.make_async_copy(ones_v, cnt_sh.at[pl.ds(0, WIN)],
                                      ssems[sp]).wait()

                @pl.when(t + 4 < cnt)
                def _():
                    issue_idx(t + 4, sp)

        return 0

    lax.fori_loop(0, (maxw + 4 + DSLOT - 1) // DSLOT, ring_body, 0)
    plsc.subcore_barrier()

    rem = N_NODES - (N_NODES // ZCHUNK) * ZCHUNK
    for c, out_hbm in ((0, out0_hbm), (1, out1_hbm)):
        @pl.when(cid == c)
        def _(out_hbm=out_hbm):
            @pl.when(sid < (N_NODES // ZCHUNK))
            def _():
                pltpu.sync_copy(cnt_sh.at[pl.ds(start, ZCHUNK)], zb_v)
                pltpu.sync_copy(zb_v, out_hbm.at[pl.ds(start, ZCHUNK)])

            @pl.when(sid == (N_NODES // ZCHUNK))
            def _():
                pltpu.sync_copy(cnt_sh.at[pl.ds(start, rem)],
                                zb_v.at[pl.ds(0, rem)])
                pltpu.sync_copy(zb_v.at[pl.ds(0, rem)],
                                out_hbm.at[pl.ds(start, rem)])


def _deg_counts(edge_index, nwin):
    k = pl.kernel(
        functools.partial(_deg_body, nwin),
        out_type=[jax.ShapeDtypeStruct((N_NODES,), jnp.float32),
                  jax.ShapeDtypeStruct((N_NODES,), jnp.float32)],
        mesh=_mesh(),
        scratch_types=(
            [pltpu.VMEM((WIN,), jnp.int32) for _ in range(8)]
            + [
                pltpu.VMEM((WIN,), jnp.float32),
                pltpu.VMEM((ZCHUNK,), jnp.float32),
                pltpu.VMEM_SHARED((N_NODES,), jnp.float32),
            ]
            + [pltpu.SemaphoreType.DMA for _ in range(16)]
        ),
    )
    return k(edge_index)


def _edge_body(nwin, hs_hbm, edges_hbm, out0_hbm, out1_hbm, *scr):
    sidx = scr[0:IBUF]
    didx = scr[IBUF:2 * IBUF]
    rows = scr[2 * IBUF:2 * IBUF + NBUF]
    acc_sh = scr[2 * IBUF + NBUF]
    isems = scr[2 * IBUF + NBUF + 1:3 * IBUF + NBUF + 1]
    gsems = scr[3 * IBUF + NBUF + 1:3 * IBUF + 2 * NBUF + 1]
    ssems = scr[3 * IBUF + 2 * NBUF + 1:3 * IBUF + 3 * NBUF + 1]

    cid = lax.axis_index("c")
    sid = lax.axis_index("s")
    wid = cid * NS + sid

    def zfill(r, _):
        for j in range(N_FEAT // 16):
            rows[0][r, pl.ds(j * 16, 16)] = jnp.zeros((16,), jnp.float32)
        return 0

    lax.fori_loop(0, WIN, zfill, 0)

    nchunk = N_NODES // WIN
    tail_rows = N_NODES - nchunk * WIN

    def zbody(k, _):
        ci = sid + NS * k
        r0 = ci * WIN

        @pl.when(ci < nchunk)
        def _():
            pltpu.sync_copy(rows[0], acc_sh.at[pl.ds(r0, WIN)])

        @pl.when(ci == nchunk)
        def _():
            pltpu.sync_copy(rows[0].at[pl.ds(0, tail_rows)],
                            acc_sh.at[pl.ds(r0, tail_rows)])

        return 0

    lax.fori_loop(0, (nchunk + NS) // NS, zbody, 0)

    lo = (nwin * wid) // NW
    hi = (nwin * (wid + 1)) // NW
    cnt = hi - lo

    def issue_idx(t, s):
        base = (lo + t) * WIN
        pltpu.async_copy(edges_hbm.at[0, pl.ds(base, WIN)], sidx[s], isems[s])
        pltpu.async_copy(edges_hbm.at[1, pl.ds(base, WIN)], didx[s], isems[s])

    def wait_idx(s):
        pltpu.make_async_copy(edges_hbm.at[0, pl.ds(0, WIN)], sidx[s],
                              isems[s]).wait()
        pltpu.make_async_copy(edges_hbm.at[0, pl.ds(0, WIN)], didx[s],
                              isems[s]).wait()

    for tt in range(IBUF):
        @pl.when(tt < cnt)
        def _(tt=tt):
            issue_idx(tt, tt)

    plsc.subcore_barrier()

    wait_idx(0)
    pltpu.async_copy(hs_hbm.at[sidx[0]], rows[0], gsems[0])

    def hex_body(g, _):
        for u in range(IBUF):
            t = 6 * g + u
            rm1 = (u + 2) % NBUF
            rm2 = (u + 1) % NBUF
            im1 = (u + 5) % IBUF
            ip1 = (u + 1) % IBUF

            @pl.when(jnp.logical_and(t >= 1, t - 1 < cnt))
            def _():
                pltpu.make_async_copy(hs_hbm.at[pl.ds(0, WIN)], rows[rm1],
                                      gsems[rm1]).wait()
                pltpu.async_copy(rows[rm1], acc_sh.at[didx[im1]],
                                 ssems[rm1], add=True)

                @pl.when(t + 5 < cnt)
                def _():
                    issue_idx(t + 5, im1)

            @pl.when(jnp.logical_and(t >= 2, t - 2 < cnt))
            def _():
                pltpu.make_async_copy(rows[rm2], acc_sh.at[pl.ds(0, WIN)],
                                      ssems[rm2]).wait()

            @pl.when(t + 1 < cnt)
            def _():
                wait_idx(ip1)
                pltpu.async_copy(hs_hbm.at[sidx[ip1]], rows[rm2], gsems[rm2])

        return 0

    lax.fori_loop(0, ((nwin + NW - 1) // NW + 2 + 5) // 6, hex_body, 0)
    plsc.subcore_barrier()

    for c, out_hbm in ((0, out0_hbm), (1, out1_hbm)):
        @pl.when(cid == c)
        def _(out_hbm=out_hbm):
            def wbody(k, _):
                ci = sid + NS * k
                r0 = ci * WIN

                @pl.when(ci < nchunk)
                def _():
                    pltpu.sync_copy(acc_sh.at[pl.ds(r0, WIN)], rows[0])
                    pltpu.sync_copy(rows[0], out_hbm.at[pl.ds(r0, WIN)])

                @pl.when(ci == nchunk)
                def _():
                    pltpu.sync_copy(acc_sh.at[pl.ds(r0, tail_rows)],
                                    rows[1].at[pl.ds(0, tail_rows)])
                    pltpu.sync_copy(rows[1].at[pl.ds(0, tail_rows)],
                                    out_hbm.at[pl.ds(r0, tail_rows)])

                return 0

            lax.fori_loop(0, (nchunk + NS) // NS, wbody, 0)


def _edge_pass(hs, edge_index, nwin):
    k = pl.kernel(
        functools.partial(_edge_body, nwin),
        out_type=[jax.ShapeDtypeStruct((N_NODES, N_FEAT), jnp.float32),
                  jax.ShapeDtypeStruct((N_NODES, N_FEAT), jnp.float32)],
        mesh=_mesh(),
        scratch_types=(
            [pltpu.VMEM((WIN,), jnp.int32) for _ in range(2 * IBUF)]
            + [pltpu.VMEM((WIN, N_FEAT), jnp.float32) for _ in range(NBUF)]
            + [pltpu.VMEM_SHARED((N_NODES, N_FEAT), jnp.float32)]
            + [pltpu.SemaphoreType.DMA for _ in range(IBUF + 2 * NBUF)]
        ),
    )
    return k(hs, edge_index)


_RB = 2000


def _dinv_from_counts(c0_ref, c1_ref):
    deg = c0_ref[...] + c1_ref[...] + 1.0
    return lax.rsqrt(deg)


def _b1_body(x_ref, w_ref, c0_ref, c1_ref, o_ref):
    dinv = _dinv_from_counts(c0_ref, c1_ref)
    h = jnp.dot(x_ref[...], w_ref[...], preferred_element_type=jnp.float32,
                precision=lax.Precision.HIGHEST)
    o_ref[...] = h * dinv


def _matmul_scale(x, W, c0, c1):
    return pl.pallas_call(
        _b1_body,
        grid=(N_NODES // _RB,),
        in_specs=[
            pl.BlockSpec((_RB, N_FEAT), lambda i: (i, 0)),
            pl.BlockSpec((N_FEAT, N_FEAT), lambda i: (0, 0)),
            pl.BlockSpec((_RB, 1), lambda i: (i, 0)),
            pl.BlockSpec((_RB, 1), lambda i: (i, 0)),
        ],
        out_specs=pl.BlockSpec((_RB, N_FEAT), lambda i: (i, 0)),
        out_shape=jax.ShapeDtypeStruct((N_NODES, N_FEAT), jnp.float32),
    )(x, W, c0, c1)


def _b2_body(a0_ref, a1_ref, hs_ref, c0_ref, c1_ref, b_ref, w_ref, o_ref):
    dinv = _dinv_from_counts(c0_ref, c1_ref)
    s = a0_ref[...] + a1_ref[...] + hs_ref[...]
    o = jax.nn.relu(dinv * s + b_ref[...])
    h2 = jnp.dot(o, w_ref[...], preferred_element_type=jnp.float32,
                 precision=lax.Precision.HIGHEST)
    o_ref[...] = h2 * dinv


def _conv_finish_matmul(a0, a1, hs, c0, c1, b2d, W):
    return pl.pallas_call(
        _b2_body,
        grid=(N_NODES // _RB,),
        in_specs=[
            pl.BlockSpec((_RB, N_FEAT), lambda i: (i, 0)),
            pl.BlockSpec((_RB, N_FEAT), lambda i: (i, 0)),
            pl.BlockSpec((_RB, N_FEAT), lambda i: (i, 0)),
            pl.BlockSpec((_RB, 1), lambda i: (i, 0)),
            pl.BlockSpec((_RB, 1), lambda i: (i, 0)),
            pl.BlockSpec((1, N_FEAT), lambda i: (0, 0)),
            pl.BlockSpec((N_FEAT, N_FEAT), lambda i: (0, 0)),
        ],
        out_specs=pl.BlockSpec((_RB, N_FEAT), lambda i: (i, 0)),
        out_shape=jax.ShapeDtypeStruct((N_NODES, N_FEAT), jnp.float32),
    )(a0, a1, hs, c0, c1, b2d, W)


def _b3_body(a0_ref, a1_ref, hs_ref, c0_ref, c1_ref, b_ref, w1_ref, bl1_ref,
             w2_ref, bl2_ref, o_ref, sums_ref):
    dinv = _dinv_from_counts(c0_ref, c1_ref)
    s = a0_ref[...] + a1_ref[...] + hs_ref[...]
    o = jax.nn.relu(dinv * s + b_ref[...])
    part = jnp.sum(o, axis=0, keepdims=True)

    @pl.when(pl.program_id(0) == 0)
    def _():
        sums_ref[...] = part

    @pl.when(pl.program_id(0) != 0)
    def _():
        sums_ref[...] += part

    @pl.when(pl.program_id(0) == pl.num_programs(0) - 1)
    def _():
        pooled = sums_ref[...] * (1.0 / N_NODES)
        y = jax.nn.relu(
            jnp.dot(pooled, w1_ref[...], preferred_element_type=jnp.float32,
                    precision=lax.Precision.HIGHEST) + bl1_ref[...])
        z = jax.nn.relu(
            jnp.dot(y, w2_ref[...], preferred_element_type=jnp.float32,
                    precision=lax.Precision.HIGHEST) + bl2_ref[...])
        o_ref[...] = z


def _conv_finish_pool_head(a0, a1, hs, c0, c1, b2d, Wl1, bl1_2d, Wl2, bl2_2d):
    return pl.pallas_call(
        _b3_body,
        grid=(N_NODES // _RB,),
        in_specs=[
            pl.BlockSpec((_RB, N_FEAT), lambda i: (i, 0)),
            pl.BlockSpec((_RB, N_FEAT), lambda i: (i, 0)),
            pl.BlockSpec((_RB, N_FEAT), lambda i: (i, 0)),
            pl.BlockSpec((_RB, 1), lambda i: (i, 0)),
            pl.BlockSpec((_RB, 1), lambda i: (i, 0)),
            pl.BlockSpec((1, N_FEAT), lambda i: (0, 0)),
            pl.BlockSpec((N_FEAT, N_FEAT), lambda i: (0, 0)),
            pl.BlockSpec((1, N_FEAT), lambda i: (0, 0)),
            pl.BlockSpec((N_FEAT, 1), lambda i: (0, 0)),
            pl.BlockSpec((1, 1), lambda i: (0, 0)),
        ],
        out_specs=pl.BlockSpec((1, 1), lambda i: (0, 0)),
        out_shape=jax.ShapeDtypeStruct((1, 1), jnp.float32),
        scratch_shapes=[pltpu.VMEM((1, N_FEAT), jnp.float32)],
    )(a0, a1, hs, c0, c1, b2d, Wl1, bl1_2d, Wl2, bl2_2d)


def kernel(x, edge_index, W1, b1, W2, b2, Wl1, bl1, Wl2, bl2):
    nwin = edge_index.shape[1] // WIN
    cnt0, cnt1 = _deg_counts(edge_index, nwin)
    c0 = cnt0.reshape(N_NODES, 1)
    c1 = cnt1.reshape(N_NODES, 1)
    hs1 = _matmul_scale(x, W1, c0, c1)
    a0, a1 = _edge_pass(hs1, edge_index, nwin)
    hs2 = _conv_finish_matmul(a0, a1, hs1, c0, c1, b1.reshape(1, -1), W2)
    a0, a1 = _edge_pass(hs2, edge_index, nwin)
    out = _conv_finish_pool_head(a0, a1, hs2, c0, c1, b2.reshape(1, -1),
                                 Wl1, bl1.reshape(1, -1), Wl2,
                                 bl2.reshape(1, 1))
    return out


# speedup vs baseline: 36.7667x; 1.0117x over previous
import functools

import jax
import jax.numpy as jnp
from jax import lax
from jax.experimental import pallas as pl
from jax.experimental.pallas import tpu as pltpu
from jax.experimental.pallas import tpu_sc as plsc

N_NODES = 10000
N_FEAT = 128
WIN = 128
NC, NS = 2, 16
NW = NC * NS
ZCHUNK = 640
NBUF = 3
IBUF = 6

def _mesh():
    return plsc.VectorSubcoreMesh(
        core_axis_name="c", subcore_axis_name="s", num_cores=NC, num_subcores=NS
    )


def _deg_body(nwin, edges_hbm, out0_hbm, out1_hbm, *scr):
    DSLOT = 8
    didx = scr[0:DSLOT]
    ones_v = scr[DSLOT]
    zb_v = scr[DSLOT + 1]
    cnt_sh = scr[DSLOT + 2]
    isems = scr[DSLOT + 3:2 * DSLOT + 3]
    ssems = scr[2 * DSLOT + 3:3 * DSLOT + 3]

    cid = lax.axis_index("c")
    sid = lax.axis_index("s")
    wid = cid * NS + sid

    for i in range(WIN // 16):
        ones_v[pl.ds(i * 16, 16)] = jnp.full((16,), 1.0, jnp.float32)
    for i in range(ZCHUNK // 16):
        zb_v[pl.ds(i * 16, 16)] = jnp.zeros((16,), jnp.float32)

    start = sid * ZCHUNK

    @pl.when(sid < (N_NODES // ZCHUNK))
    def _():
        pltpu.sync_copy(zb_v, cnt_sh.at[pl.ds(start, ZCHUNK)])

    @pl.when(sid == (N_NODES // ZCHUNK))
    def _():
        rem = N_NODES - (N_NODES // ZCHUNK) * ZCHUNK
        pltpu.sync_copy(zb_v.at[pl.ds(0, rem)], cnt_sh.at[pl.ds(start, rem)])

    lo = (nwin * wid) // NW
    hi = (nwin * (wid + 1)) // NW
    cnt = hi - lo

    def issue_idx(t, s):
        pltpu.async_copy(edges_hbm.at[1, pl.ds((lo + t) * WIN, WIN)],
                         didx[s], isems[s])

    for ss in range(DSLOT):
        @pl.when(ss < cnt)
        def _(ss=ss):
            issue_idx(ss, ss)

    plsc.subcore_barrier()

    maxw = (nwin + NW - 1) // NW

    def ring_body(g, _):
        for u in range(DSLOT):
            t = DSLOT * g + u
            sp = (u + 4) % DSLOT

            @pl.when(t < cnt)
            def _():
                pltpu.make_async_copy(edges_hbm.at[1, pl.ds(0, WIN)],
                                      didx[u], isems[u]).wait()
                pltpu.async_copy(ones_v, cnt_sh.at[didx[u]], ssems[u],
                                 add=True)

            @pl.when(jnp.logical_and(t >= 4, t - 4 < cnt))
            def _():
                pltpu.make_async_copy(ones_v, cnt_sh.at[pl.ds(0, WIN)],
                                      ssems[sp]).wait()

                @pl.when(t + 4 < cnt)
                def _():
                    issue_idx(t + 4, sp)

        return 0

    lax.fori_loop(0, (maxw + 4 + DSLOT - 1) // DSLOT, ring_body, 0)
    plsc.subcore_barrier()

    rem = N_NODES - (N_NODES // ZCHUNK) * ZCHUNK
    for c, out_hbm in ((0, out0_hbm), (1, out1_hbm)):
        @pl.when(cid == c)
        def _(out_hbm=out_hbm):
            @pl.when(sid < (N_NODES // ZCHUNK))
            def _():
                pltpu.sync_copy(cnt_sh.at[pl.ds(start, ZCHUNK)], zb_v)
                pltpu.sync_copy(zb_v, out_hbm.at[pl.ds(start, ZCHUNK)])

            @pl.when(sid == (N_NODES // ZCHUNK))
            def _():
                pltpu.sync_copy(cnt_sh.at[pl.ds(start, rem)],
                                zb_v.at[pl.ds(0, rem)])
                pltpu.sync_copy(zb_v.at[pl.ds(0, rem)],
                                out_hbm.at[pl.ds(start, rem)])


def _deg_counts(edge_index, nwin):
    k = pl.kernel(
        functools.partial(_deg_body, nwin),
        out_type=[jax.ShapeDtypeStruct((N_NODES,), jnp.float32),
                  jax.ShapeDtypeStruct((N_NODES,), jnp.float32)],
        mesh=_mesh(),
        scratch_types=(
            [pltpu.VMEM((WIN,), jnp.int32) for _ in range(8)]
            + [
                pltpu.VMEM((WIN,), jnp.float32),
                pltpu.VMEM((ZCHUNK,), jnp.float32),
                pltpu.VMEM_SHARED((N_NODES,), jnp.float32),
            ]
            + [pltpu.SemaphoreType.DMA for _ in range(16)]
        ),
    )
    return k(edge_index)


def _edge_body(nwin, hs_hbm, edges_hbm, out0_hbm, out1_hbm, *scr):
    sidx = scr[0:IBUF]
    didx = scr[IBUF:2 * IBUF]
    rows = scr[2 * IBUF:2 * IBUF + NBUF]
    acc_sh = scr[2 * IBUF + NBUF]
    isems = scr[2 * IBUF + NBUF + 1:3 * IBUF + NBUF + 1]
    gsems = scr[3 * IBUF + NBUF + 1:3 * IBUF + 2 * NBUF + 1]
    ssems = scr[3 * IBUF + 2 * NBUF + 1:3 * IBUF + 3 * NBUF + 1]

    cid = lax.axis_index("c")
    sid = lax.axis_index("s")
    wid = cid * NS + sid

    def zfill(r, _):
        for j in range(N_FEAT // 16):
            rows[0][r, pl.ds(j * 16, 16)] = jnp.zeros((16,), jnp.float32)
        return 0

    lax.fori_loop(0, WIN, zfill, 0)

    nchunk = N_NODES // WIN
    tail_rows = N_NODES - nchunk * WIN

    def zbody(k, _):
        ci = sid + NS * k
        r0 = ci * WIN

        @pl.when(ci < nchunk)
        def _():
            pltpu.sync_copy(rows[0], acc_sh.at[pl.ds(r0, WIN)])

        @pl.when(ci == nchunk)
        def _():
            pltpu.sync_copy(rows[0].at[pl.ds(0, tail_rows)],
                            acc_sh.at[pl.ds(r0, tail_rows)])

        return 0

    lax.fori_loop(0, (nchunk + NS) // NS, zbody, 0)

    lo = (nwin * wid) // NW
    hi = (nwin * (wid + 1)) // NW
    cnt = hi - lo

    def issue_idx(t, s):
        base = (lo + t) * WIN
        pltpu.async_copy(edges_hbm.at[0, pl.ds(base, WIN)], sidx[s], isems[s])
        pltpu.async_copy(edges_hbm.at[1, pl.ds(base, WIN)], didx[s], isems[s])

    def wait_idx(s):
        pltpu.make_async_copy(edges_hbm.at[0, pl.ds(0, WIN)], sidx[s],
                              isems[s]).wait()
        pltpu.make_async_copy(edges_hbm.at[0, pl.ds(0, WIN)], didx[s],
                              isems[s]).wait()

    for tt in range(IBUF):
        @pl.when(tt < cnt)
        def _(tt=tt):
            issue_idx(tt, tt)

    plsc.subcore_barrier()

    wait_idx(0)
    pltpu.async_copy(hs_hbm.at[sidx[0]], rows[0], gsems[0])

    def hex_body(g, _):
        for u in range(IBUF):
            t = 6 * g + u
            rm1 = (u + 2) % NBUF
            rm2 = (u + 1) % NBUF
            im1 = (u + 5) % IBUF
            ip1 = (u + 1) % IBUF

            @pl.when(jnp.logical_and(t >= 1, t - 1 < cnt))
            def _():
                pltpu.make_async_copy(hs_hbm.at[pl.ds(0, WIN)], rows[rm1],
                                      gsems[rm1]).wait()
                pltpu.async_copy(rows[rm1], acc_sh.at[didx[im1]],
                                 ssems[rm1], add=True)

                @pl.when(t + 5 < cnt)
                def _():
                    issue_idx(t + 5, im1)

            @pl.when(jnp.logical_and(t >= 2, t - 2 < cnt))
            def _():
                pltpu.make_async_copy(rows[rm2], acc_sh.at[pl.ds(0, WIN)],
                                      ssems[rm2]).wait()

            @pl.when(t + 1 < cnt)
            def _():
                wait_idx(ip1)
                pltpu.async_copy(hs_hbm.at[sidx[ip1]], rows[rm2], gsems[rm2])

        return 0

    lax.fori_loop(0, ((nwin + NW - 1) // NW + 2 + 5) // 6, hex_body, 0)
    plsc.subcore_barrier()

    for c, out_hbm in ((0, out0_hbm), (1, out1_hbm)):
        @pl.when(cid == c)
        def _(out_hbm=out_hbm):
            def wbody(k, _):
                ci = sid + NS * k
                r0 = ci * WIN

                @pl.when(ci < nchunk)
                def _():
                    pltpu.sync_copy(acc_sh.at[pl.ds(r0, WIN)], rows[0])
                    pltpu.sync_copy(rows[0], out_hbm.at[pl.ds(r0, WIN)])

                @pl.when(ci == nchunk)
                def _():
                    pltpu.sync_copy(acc_sh.at[pl.ds(r0, tail_rows)],
                                    rows[1].at[pl.ds(0, tail_rows)])
                    pltpu.sync_copy(rows[1].at[pl.ds(0, tail_rows)],
                                    out_hbm.at[pl.ds(r0, tail_rows)])

                return 0

            lax.fori_loop(0, (nchunk + NS) // NS, wbody, 0)


def _edge_pass(hs, edge_index, nwin):
    k = pl.kernel(
        functools.partial(_edge_body, nwin),
        out_type=[jax.ShapeDtypeStruct((N_NODES, N_FEAT), jnp.float32),
                  jax.ShapeDtypeStruct((N_NODES, N_FEAT), jnp.float32)],
        mesh=_mesh(),
        scratch_types=(
            [pltpu.VMEM((WIN,), jnp.int32) for _ in range(2 * IBUF)]
            + [pltpu.VMEM((WIN, N_FEAT), jnp.float32) for _ in range(NBUF)]
            + [pltpu.VMEM_SHARED((N_NODES, N_FEAT), jnp.float32)]
            + [pltpu.SemaphoreType.DMA for _ in range(IBUF + 2 * NBUF)]
        ),
    )
    return k(hs, edge_index)


_RB = 2000


def _dinv_from_counts(c0_ref, c1_ref):
    deg = c0_ref[...] + c1_ref[...] + 1.0
    return 1.0 / jnp.sqrt(deg)


def _b1_body(x_ref, w_ref, c0_ref, c1_ref, o_ref):
    dinv = _dinv_from_counts(c0_ref, c1_ref)
    h = jnp.dot(x_ref[...], w_ref[...], preferred_element_type=jnp.float32)
    o_ref[...] = h * dinv


def _matmul_scale(x, W, c0, c1):
    return pl.pallas_call(
        _b1_body,
        grid=(N_NODES // _RB,),
        in_specs=[
            pl.BlockSpec((_RB, N_FEAT), lambda i: (i, 0)),
            pl.BlockSpec((N_FEAT, N_FEAT), lambda i: (0, 0)),
            pl.BlockSpec((_RB, 1), lambda i: (i, 0)),
            pl.BlockSpec((_RB, 1), lambda i: (i, 0)),
        ],
        out_specs=pl.BlockSpec((_RB, N_FEAT), lambda i: (i, 0)),
        out_shape=jax.ShapeDtypeStruct((N_NODES, N_FEAT), jnp.float32),
    )(x, W, c0, c1)


def _b2_body(a0_ref, a1_ref, hs_ref, c0_ref, c1_ref, b_ref, w_ref, o_ref):
    dinv = _dinv_from_counts(c0_ref, c1_ref)
    s = a0_ref[...] + a1_ref[...] + hs_ref[...]
    o = jax.nn.relu(dinv * s + b_ref[...])
    h2 = jnp.dot(o, w_ref[...], preferred_element_type=jnp.float32)
    o_ref[...] = h2 * dinv


def _conv_finish_matmul(a0, a1, hs, c0, c1, b2d, W):
    return pl.pallas_call(
        _b2_body,
        grid=(N_NODES // _RB,),
        in_specs=[
            pl.BlockSpec((_RB, N_FEAT), lambda i: (i, 0)),
            pl.BlockSpec((_RB, N_FEAT), lambda i: (i, 0)),
            pl.BlockSpec((_RB, N_FEAT), lambda i: (i, 0)),
            pl.BlockSpec((_RB, 1), lambda i: (i, 0)),
            pl.BlockSpec((_RB, 1), lambda i: (i, 0)),
            pl.BlockSpec((1, N_FEAT), lambda i: (0, 0)),
            pl.BlockSpec((N_FEAT, N_FEAT), lambda i: (0, 0)),
        ],
        out_specs=pl.BlockSpec((_RB, N_FEAT), lambda i: (i, 0)),
        out_shape=jax.ShapeDtypeStruct((N_NODES, N_FEAT), jnp.float32),
    )(a0, a1, hs, c0, c1, b2d, W)


def _b3_body(a0_ref, a1_ref, hs_ref, c0_ref, c1_ref, b_ref, w1_ref, bl1_ref,
             w2_ref, bl2_ref, o_ref, sums_ref):
    dinv = _dinv_from_counts(c0_ref, c1_ref)
    s = a0_ref[...] + a1_ref[...] + hs_ref[...]
    o = jax.nn.relu(dinv * s + b_ref[...])
    part = jnp.sum(o, axis=0, keepdims=True)

    @pl.when(pl.program_id(0) == 0)
    def _():
        sums_ref[...] = part

    @pl.when(pl.program_id(0) != 0)
    def _():
        sums_ref[...] += part

    @pl.when(pl.program_id(0) == pl.num_programs(0) - 1)
    def _():
        pooled = sums_ref[...] * (1.0 / N_NODES)
        y = jax.nn.relu(
            jnp.dot(pooled, w1_ref[...], preferred_element_type=jnp.float32) + bl1_ref[...])
        z = jax.nn.relu(
            jnp.dot(y, w2_ref[...], preferred_element_type=jnp.float32) + bl2_ref[...])
        o_ref[...] = z


def _conv_finish_pool_head(a0, a1, hs, c0, c1, b2d, Wl1, bl1_2d, Wl2, bl2_2d):
    return pl.pallas_call(
        _b3_body,
        grid=(N_NODES // _RB,),
        in_specs=[
            pl.BlockSpec((_RB, N_FEAT), lambda i: (i, 0)),
            pl.BlockSpec((_RB, N_FEAT), lambda i: (i, 0)),
            pl.BlockSpec((_RB, N_FEAT), lambda i: (i, 0)),
            pl.BlockSpec((_RB, 1), lambda i: (i, 0)),
            pl.BlockSpec((_RB, 1), lambda i: (i, 0)),
            pl.BlockSpec((1, N_FEAT), lambda i: (0, 0)),
            pl.BlockSpec((N_FEAT, N_FEAT), lambda i: (0, 0)),
            pl.BlockSpec((1, N_FEAT), lambda i: (0, 0)),
            pl.BlockSpec((N_FEAT, 1), lambda i: (0, 0)),
            pl.BlockSpec((1, 1), lambda i: (0, 0)),
        ],
        out_specs=pl.BlockSpec((1, 1), lambda i: (0, 0)),
        out_shape=jax.ShapeDtypeStruct((1, 1), jnp.float32),
        scratch_shapes=[pltpu.VMEM((1, N_FEAT), jnp.float32)],
    )(a0, a1, hs, c0, c1, b2d, Wl1, bl1_2d, Wl2, bl2_2d)


def kernel(x, edge_index, W1, b1, W2, b2, Wl1, bl1, Wl2, bl2):
    nwin = edge_index.shape[1] // WIN
    cnt0, cnt1 = _deg_counts(edge_index, nwin)
    c0 = cnt0.reshape(N_NODES, 1)
    c1 = cnt1.reshape(N_NODES, 1)
    hs1 = _matmul_scale(x, W1, c0, c1)
    a0, a1 = _edge_pass(hs1, edge_index, nwin)
    hs2 = _conv_finish_matmul(a0, a1, hs1, c0, c1, b1.reshape(1, -1), W2)
    a0, a1 = _edge_pass(hs2, edge_index, nwin)
    out = _conv_finish_pool_head(a0, a1, hs2, c0, c1, b2.reshape(1, -1),
                                 Wl1, bl1.reshape(1, -1), Wl2,
                                 bl2.reshape(1, 1))
    return out
